# baseline (speedup 1.0000x reference)
import functools

import jax
import jax.numpy as jnp
from jax import lax
from jax.experimental import pallas as pl
from jax.experimental.pallas import tpu as pltpu
from jax.experimental.pallas import tpu_sc as plsc

F32 = jnp.float32
I32 = jnp.int32

_CH = 64
_NW = 32
_STRIDE = _NW * _CH
_ROW_PAD = 1024
_SPMEM_BUDGET = 4 * 1024 * 1024


def _ceil_to(x, m):
    return -(-x // m) * m


def _feat_chunks(d, n_pad):
    max_dc = min(368, (7_200_000 // (4 * n_pad + 8192)) // 16 * 16)
    nc = -(-d // max_dc)
    base = (d // nc) // 16 * 16
    sizes = [base] * (nc - 1) + [d - base * (nc - 1)]
    assert all(16 <= s <= max_dc and s % 16 == 0 for s in sizes), (d, n_pad, sizes)
    return tuple(sizes)


@functools.lru_cache(maxsize=None)
def _sc_scatter_builder(nx, dc, nnz_pad, n_pad, mode):
    nchunk = nnz_pad // (_NW * _CH)
    rows_per_tile = n_pad // 16
    n_copies = rows_per_tile // _CH
    ncol = dc // 16
    mesh = plsc.VectorSubcoreMesh(core_axis_name="c", subcore_axis_name="s")
    assert mode == "ones" or nchunk % 2 == 0, (mode, nchunk)

    scratch = []
    if mode != "ones":
        scratch.append(pltpu.VMEM((2, _CH), I32))
    scratch.append(pltpu.VMEM((2, _CH), I32))
    if mode == "gs":
        scratch.append(pltpu.VMEM((2, _CH, 16), F32))
    scratch += [
        pltpu.VMEM((2, _CH, dc), F32),
        pltpu.VMEM_SHARED((n_pad, dc), F32),
        pltpu.SemaphoreType.DMA,
        pltpu.SemaphoreType.DMA,
    ]

    def kern(*args):
        if mode == "gs":
            (x_hbm, gidx_hbm, dst_hbm, w_hbm, out_hbm,
             gidx_v, dst_v, w_v, rows_v, acc, sem0, sem1) = args
        elif mode == "g":
            (x_hbm, gidx_hbm, dst_hbm, out_hbm,
             gidx_v, dst_v, rows_v, acc, sem0, sem1) = args
        else:
            (dst_hbm, out_hbm, dst_v, rows_v, acc, sem0, sem1) = args
        sems = (sem0, sem1)
        c = lax.axis_index("c")
        s = lax.axis_index("s")
        wid = s * 2 + c
        t0 = s * rows_per_tile

        def fill(val):
            vec = jnp.full((16,), val, F32)

            def row(r, carry):
                for b in range(2):
                    for k in range(ncol):
                        rows_v[b, r, pl.ds(16 * k, 16)] = vec
                return carry

            lax.fori_loop(0, _CH, row, 0)

        fill(0.0)
        for j in range(n_copies):
            pltpu.sync_copy(rows_v.at[0], acc.at[pl.ds(t0 + j * _CH, _CH)])
        plsc.subcore_barrier()

        if mode == "ones":
            fill(1.0)

            def chunk(ci, carry):
                base = (wid * nchunk + ci) * _CH
                pltpu.sync_copy(dst_hbm.at[pl.ds(base, _CH)], dst_v.at[0])
                pltpu.sync_copy(rows_v.at[0], acc.at[dst_v.at[0]], add=True)
                return carry

            lax.fori_loop(0, nchunk, chunk, 0)
        else:
            def stage(ci, b):
                base = (wid * nchunk + ci) * _CH
                pltpu.sync_copy(gidx_hbm.at[pl.ds(base, _CH)], gidx_v.at[b])
                pltpu.sync_copy(dst_hbm.at[pl.ds(base, _CH)], dst_v.at[b])
                if mode == "gs":
                    pltpu.sync_copy(w_hbm.at[pl.ds(base, _CH)], w_v.at[b])
                pltpu.async_copy(x_hbm.at[gidx_v.at[b]], rows_v.at[b], sems[b])

            stage(0, 0)

            def pair(pi, carry):
                for ph in range(2):
                    b = ph
                    ci = pi * 2 + ph
                    pltpu.make_async_copy(
                        x_hbm.at[gidx_v.at[b]], rows_v.at[b], sems[b]).wait()

                    @pl.when(ci + 1 < nchunk)
                    def _():
                        stage(ci + 1, 1 - b)

                    if mode == "gs":
                        def sgrp(g, carry2):
                            for r8 in range(8):
                                r = g * 8 + r8
                                ws = w_v[b, r, pl.ds(0, 16)]
                                for k in range(ncol):
                                    sl = pl.ds(16 * k, 16)
                                    rows_v[b, r, sl] = rows_v[b, r, sl] * ws
                            return carry2

                        lax.fori_loop(0, _CH // 8, sgrp, 0)
                    pltpu.sync_copy(rows_v.at[b], acc.at[dst_v.at[b]], add=True)
                return carry

            lax.fori_loop(0, nchunk // 2, pair, 0)
        plsc.subcore_barrier()
        for j in range(n_copies):
            sl = pl.ds(t0 + j * _CH, _CH)
            pltpu.sync_copy(acc.at[sl], out_hbm.at[c, sl])

    return pl.kernel(
        kern,
        out_type=jax.ShapeDtypeStruct((2, n_pad, dc), F32),
        mesh=mesh,
        scratch_types=scratch,
        compiler_params=pltpu.CompilerParams(use_tc_tiling_on_sc=False),
    )


@functools.lru_cache(maxsize=None)
def _sc_diff_builder(nx, dc, e_pad):
    nchunk = e_pad // (_NW * _CH)
    ncol = dc // 16
    mesh = plsc.VectorSubcoreMesh(core_axis_name="c", subcore_axis_name="s")
    assert nchunk % 2 == 0, nchunk

    def kern(x_hbm, src_hbm, dst_hbm, out_hbm, si_v, di_v, ra_v, rb_v,
             sem_a0, sem_a1, sem_b0, sem_b1, sem_w):
        sems_a = (sem_a0, sem_a1)
        sems_b = (sem_b0, sem_b1)
        c = lax.axis_index("c")
        s = lax.axis_index("s")
        wid = s * 2 + c

        def stage(ci, b):
            base = (wid * nchunk + ci) * _CH
            pltpu.sync_copy(dst_hbm.at[pl.ds(base, _CH)], di_v.at[b])
            pltpu.sync_copy(src_hbm.at[pl.ds(base, _CH)], si_v.at[b])
            pltpu.async_copy(x_hbm.at[di_v.at[b]], ra_v.at[b], sems_a[b])
            pltpu.async_copy(x_hbm.at[si_v.at[b]], rb_v.at[b], sems_b[b])

        stage(0, 0)

        def pair(pi, carry):
            for ph in range(2):
                b = ph
                ci = pi * 2 + ph
                pltpu.make_async_copy(
                    x_hbm.at[di_v.at[b]], ra_v.at[b], sems_a[b]).wait()
                pltpu.make_async_copy(
                    x_hbm.at[si_v.at[b]], rb_v.at[b], sems_b[b]).wait()

                @pl.when(ci + 1 < nchunk)
                def _():
                    stage(ci + 1, 1 - b)

                def grp(g, carry2):
                    for r8 in range(8):
                        r = g * 8 + r8
                        for k in range(ncol):
                            sl = pl.ds(16 * k, 16)
                            ra_v[b, r, sl] = ra_v[b, r, sl] - rb_v[b, r, sl]
                    return carry2

                lax.fori_loop(0, _CH // 8, grp, 0)
                base = (wid * nchunk + ci) * _CH
                pltpu.async_copy(ra_v.at[b], out_hbm.at[pl.ds(base, _CH)],
                                 sem_w).wait()
            return carry

        lax.fori_loop(0, nchunk // 2, pair, 0)

    return pl.kernel(
        kern,
        out_type=jax.ShapeDtypeStruct((e_pad, dc), F32),
        mesh=mesh,
        scratch_types=[
            pltpu.VMEM((2, _CH), I32), pltpu.VMEM((2, _CH), I32),
            pltpu.VMEM((2, _CH, dc), F32), pltpu.VMEM((2, _CH, dc), F32),
            pltpu.SemaphoreType.DMA, pltpu.SemaphoreType.DMA,
            pltpu.SemaphoreType.DMA, pltpu.SemaphoreType.DMA,
            pltpu.SemaphoreType.DMA,
        ],
        compiler_params=pltpu.CompilerParams(use_tc_tiling_on_sc=False),
    )


def _pad_entries(arr, nnz_pad, value):
    n = arr.shape[0]
    if n == nnz_pad:
        return arr
    return jnp.concatenate([arr, jnp.full((nnz_pad - n,), value, arr.dtype)])


def _sc_scatter(x, gidx, dst, w, n_out):
    nnz = dst.shape[0]
    nnz_pad = _ceil_to(nnz, 2 * _STRIDE)
    n_pad = _ceil_to(n_out + 1, _ROW_PAD)
    gidx_p = _pad_entries(gidx, nnz_pad, 0)
    dst_p = _pad_entries(dst, nnz_pad, n_out)
    w_p = None
    if w is not None:
        w_p = jnp.broadcast_to(_pad_entries(w, nnz_pad, 0.0)[:, None],
                               (nnz_pad, 16))
    d = x.shape[1]
    parts = []
    c0 = 0
    chunks = _feat_chunks(d, n_pad)
    for dc in chunks:
        xc = lax.slice_in_dim(x, c0, c0 + dc, axis=1)
        if w is None:
            fn = _sc_scatter_builder(x.shape[0], dc, nnz_pad, n_pad, "g")
            parts.append(fn(xc, gidx_p, dst_p))
        else:
            fn = _sc_scatter_builder(x.shape[0], dc, nnz_pad, n_pad, "gs")
            parts.append(fn(xc, gidx_p, dst_p, w_p))
        c0 += dc
    return parts, chunks


def _sc_bincount(idx, n_out):
    nnz = idx.shape[0]
    nnz_pad = _ceil_to(nnz, _STRIDE)
    n_pad = _ceil_to(n_out + 1, _ROW_PAD)
    dst_p = _pad_entries(idx, nnz_pad, n_out)
    fn = _sc_scatter_builder(0, 16, nnz_pad, n_pad, "ones")
    return fn(dst_p)


def _row_block(n_pad, d_tot):
    br = 2048 if n_pad % 2048 == 0 else n_pad
    if d_tot >= 704 and br > 1024:
        br = 1024
    return br


@functools.lru_cache(maxsize=None)
def _mm_stats_builder(n_pad, n_true, k, f):
    br = _row_block(n_pad, k)
    grid = n_pad // br

    def body(x_ref, w_ref, y_ref, st_ref):
        y = jnp.dot(x_ref[...], w_ref[...], preferred_element_type=F32)
        y_ref[...] = y
        i = pl.program_id(0)

        @pl.when(i == 0)
        def _():
            st_ref[...] = jnp.zeros_like(st_ref)

        row = i * br + lax.broadcasted_iota(I32, (br, 1), 0)
        ym = jnp.where(row < n_true, y, 0.0)
        st_ref[0:1, :] = st_ref[0:1, :] + jnp.sum(ym, axis=0, keepdims=True)
        st_ref[1:2, :] = st_ref[1:2, :] + jnp.sum(ym * ym, axis=0, keepdims=True)

    return pl.pallas_call(
        body,
        grid=(grid,),
        in_specs=[pl.BlockSpec((br, k), lambda i: (i, 0)),
                  pl.BlockSpec((k, f), lambda i: (0, 0))],
        out_specs=[pl.BlockSpec((br, f), lambda i: (i, 0)),
                   pl.BlockSpec((8, f), lambda i: (0, 0))],
        out_shape=[jax.ShapeDtypeStruct((n_pad, f), F32),
                   jax.ShapeDtypeStruct((8, f), F32)],
    )


@functools.lru_cache(maxsize=None)
def _laguerre_builder(n_pad, n_true, d, f, chunks):
    br = _row_block(n_pad, d)
    grid = n_pad // br
    nchunks = len(chunks)

    def body(*refs):
        x_ref = refs[0]
        p_refs = refs[1:1 + nchunks]
        w0_ref, w1_ref, y_ref, st_ref = refs[1 + nchunks:]
        x = x_ref[...]
        lx = jnp.concatenate([p[0] + p[1] for p in p_refs], axis=-1) \
            if nchunks > 1 else (p_refs[0][0] + p_refs[0][1])
        y = (jnp.dot(x, w0_ref[...], preferred_element_type=F32)
             + jnp.dot(x - lx, w1_ref[...], preferred_element_type=F32))
        y_ref[...] = y
        i = pl.program_id(0)

        @pl.when(i == 0)
        def _():
            st_ref[...] = jnp.zeros_like(st_ref)

        row = i * br + lax.broadcasted_iota(I32, (br, 1), 0)
        ym = jnp.where(row < n_true, y, 0.0)
        st_ref[0:1, :] = st_ref[0:1, :] + jnp.sum(ym, axis=0, keepdims=True)
        st_ref[1:2, :] = st_ref[1:2, :] + jnp.sum(ym * ym, axis=0, keepdims=True)

    in_specs = [pl.BlockSpec((br, d), lambda i: (i, 0))]
    for dc in chunks:
        in_specs.append(pl.BlockSpec((2, br, dc), lambda i: (0, i, 0)))
    in_specs += [pl.BlockSpec((d, f), lambda i: (0, 0)),
                 pl.BlockSpec((d, f), lambda i: (0, 0))]
    return pl.pallas_call(
        body,
        grid=(grid,),
        in_specs=in_specs,
        out_specs=[pl.BlockSpec((br, f), lambda i: (i, 0)),
                   pl.BlockSpec((8, f), lambda i: (0, 0))],
        out_shape=[jax.ShapeDtypeStruct((n_pad, f), F32),
                   jax.ShapeDtypeStruct((8, f), F32)],
    )


@functools.lru_cache(maxsize=None)
def _msg_mm_builder(n_pad, d, f, chunks, use_counts, act):
    br = _row_block(n_pad, d)
    grid = n_pad // br
    nchunks = 1 if chunks is None else len(chunks)

    def body(*refs):
        x_ref = refs[0]
        p_refs = refs[1:1 + nchunks]
        rest = refs[1 + nchunks:]
        if use_counts:
            c_ref, w_ref, o_ref = rest
        else:
            w_ref, o_ref = rest
        if chunks is None:
            m = p_refs[0][...]
        else:
            m = jnp.concatenate([p[0] + p[1] for p in p_refs], axis=-1) \
                if nchunks > 1 else (p_refs[0][0] + p_refs[0][1])
        if use_counts:
            cnt = c_ref[0, :, 0:1] + c_ref[1, :, 0:1]
            m = m / (cnt + 1e-6)
        x = x_ref[...]
        y = jnp.dot(x + m, w_ref[...], preferred_element_type=F32)
        if act == "relu":
            o_ref[...] = jnp.maximum(y, 0.0)
        else:
            o_ref[...] = x * jax.nn.sigmoid(y)

    in_specs = [pl.BlockSpec((br, d), lambda i: (i, 0))]
    if chunks is None:
        in_specs.append(pl.BlockSpec((br, d), lambda i: (i, 0)))
    else:
        for dc in chunks:
            in_specs.append(pl.BlockSpec((2, br, dc), lambda i: (0, i, 0)))
    if use_counts:
        in_specs.append(pl.BlockSpec((2, br, 16), lambda i: (0, i, 0)))
    in_specs.append(pl.BlockSpec((d, f), lambda i: (0, 0)))
    return pl.pallas_call(
        body,
        grid=(grid,),
        in_specs=in_specs,
        out_specs=pl.BlockSpec((br, f), lambda i: (i, 0)),
        out_shape=jax.ShapeDtypeStruct((n_pad, f), F32),
    )


@functools.lru_cache(maxsize=None)
def _bn_relu_builder(n_pad, n_true, f):
    br = _row_block(n_pad, f)
    grid = n_pad // br
    inv_n = 1.0 / n_true

    def body(y_ref, st_ref, o_ref):
        mu = st_ref[0:1, :] * inv_n
        var = st_ref[1:2, :] * inv_n - mu * mu
        y = jnp.maximum((y_ref[...] - mu) * lax.rsqrt(var + 1e-5), 0.0)
        row = pl.program_id(0) * br + lax.broadcasted_iota(I32, (br, 1), 0)
        o_ref[...] = jnp.where(row < n_true, y, 0.0)

    return pl.pallas_call(
        body,
        grid=(grid,),
        in_specs=[pl.BlockSpec((br, f), lambda i: (i, 0)),
                  pl.BlockSpec((8, f), lambda i: (0, 0))],
        out_specs=pl.BlockSpec((br, f), lambda i: (i, 0)),
        out_shape=jax.ShapeDtypeStruct((n_pad, f), F32),
    )


@functools.lru_cache(maxsize=None)
def _mean_combine_builder(n_pad, dc):
    br = 2048 if n_pad % 2048 == 0 else n_pad
    grid = n_pad // br

    def body(p_ref, c_ref, o_ref):
        cnt = c_ref[0, :, 0:1] + c_ref[1, :, 0:1]
        o_ref[...] = (p_ref[0] + p_ref[1]) / jnp.maximum(cnt, 1.0)

    return pl.pallas_call(
        body,
        grid=(grid,),
        in_specs=[pl.BlockSpec((2, br, dc), lambda i: (0, i, 0)),
                  pl.BlockSpec((2, br, 16), lambda i: (0, i, 0))],
        out_specs=pl.BlockSpec((br, dc), lambda i: (i, 0)),
        out_shape=jax.ShapeDtypeStruct((n_pad, dc), F32),
    )


@functools.lru_cache(maxsize=None)
def _head_builder(k, f):
    def body(x_ref, w_ref, b_ref, o_ref):
        o_ref[...] = (jnp.dot(x_ref[...], w_ref[...], preferred_element_type=F32)
                      + b_ref[0:1, :])

    return pl.pallas_call(
        body,
        grid=(1,),
        in_specs=[pl.BlockSpec((64, k), lambda i: (0, 0)),
                  pl.BlockSpec((k, f), lambda i: (0, 0)),
                  pl.BlockSpec((8, f), lambda i: (0, 0))],
        out_specs=pl.BlockSpec((64, f), lambda i: (0, 0)),
        out_shape=jax.ShapeDtypeStruct((64, f), F32),
    )


def _bn_relu(y, st, n_true):
    return _bn_relu_builder(y.shape[0], n_true, y.shape[1])(y, st)


def _messages(x_table, inc, n_out):
    src, dst = inc[0], inc[1]
    e = src.shape[0]
    ar = jnp.arange(e, dtype=I32)
    ones = jnp.ones((e,), F32)
    gidx = jnp.concatenate([ar, ar])
    dsts = jnp.concatenate([src, dst])
    w = jnp.concatenate([-ones, ones])
    return _sc_scatter(x_table, gidx, dsts, w, n_out)


def _gather_diff(x_table, inc, n_out):
    src, dst = inc[0], inc[1]
    e = src.shape[0]
    e_pad = _ceil_to(e, 2 * _STRIDE)
    src_p = _pad_entries(src, e_pad, 0)
    dst_p = _pad_entries(dst, e_pad, 0)
    d = x_table.shape[1]
    nc = -(-d // 320)
    base = (d // nc) // 16 * 16
    sizes = [base] * (nc - 1) + [d - base * (nc - 1)]
    outs = []
    c0 = 0
    for dc in sizes:
        xc = lax.slice_in_dim(x_table, c0, c0 + dc, axis=1)
        outs.append(_sc_diff_builder(x_table.shape[0], dc, e_pad)(
            xc, src_p, dst_p))
        c0 += dc
    return jnp.concatenate(outs, axis=-1) if len(outs) > 1 else outs[0]


def _msg_mm(x, parts, chunks, counts, w, act):
    n_pad, d = x.shape
    fn = _msg_mm_builder(n_pad, d, w.shape[1], chunks, counts is not None, act)
    args = [x] + list(parts)
    if counts is not None:
        args.append(counts)
    args.append(w)
    return fn(*args)


def _laguerre_bn(x, ei, ew, w0, w1, n_true):
    n_pad, d = x.shape
    parts, chunks = _sc_scatter(x, ei[0], ei[1], ew, n_true)
    y, st = _laguerre_builder(n_pad, n_true, d, w0.shape[1], chunks)(
        x, *parts, w0, w1)
    return _bn_relu(y, st, n_true)


def _scatter_mean(x_table, n_rows, idx, counts, n_out):
    ar = jnp.arange(n_rows, dtype=I32)
    parts, chunks = _sc_scatter(x_table, ar, idx, None, n_out)
    n_pad = parts[0].shape[1]
    outs = [_mean_combine_builder(n_pad, dc)(p, counts)
            for p, dc in zip(parts, chunks)]
    return jnp.concatenate(outs, axis=-1) if len(outs) > 1 else outs[0]


def kernel(x_t, x_s, edge_weight_t, edge_weight_s, edge_weight_t1,
           edge_weight_s1, params, edge_index_t, edge_index_s, edge_index,
           edge_index_t1, edge_index_s1, edge_index1, pos_t, pos_s,
           n_batch1, s_batch1):
    p = params
    filters = [64, 128, 256, 512]
    channels = [2, 2, 2, 2]
    n0 = x_t.shape[0]
    e0 = x_s.shape[0]
    n1 = edge_index_t1.shape[1] // 3 * 0 + 2000
    e1 = 2000
    ngraph = 64
    n0_pad = _ceil_to(n0 + 1, _ROW_PAD)
    e0_pad = _ceil_to(e0 + 1, _ROW_PAD)

    kt = _ceil_to(x_t.shape[1], 128)
    ks = _ceil_to(x_s.shape[1], 128)
    xtp = jnp.pad(x_t, ((0, n0_pad - n0), (0, kt - x_t.shape[1])))
    xsp = jnp.pad(x_s, ((0, e0_pad - e0), (0, ks - x_s.shape[1])))
    wt0 = jnp.pad(p["init_Wt"], ((0, kt - p["init_Wt"].shape[0]), (0, 0)))
    ws0 = jnp.pad(p["init_Ws"], ((0, ks - p["init_Ws"].shape[0]), (0, 0)))
    y, st = _mm_stats_builder(n0_pad, n0, kt, 64)(xtp, wt0)
    xt = _bn_relu(y, st, n0)
    y, st = _mm_stats_builder(e0_pad, e0, ks, 64)(xsp, ws0)
    xs = _bn_relu(y, st, e0)

    xt0, xs0 = xt, xs
    ei_t, ew_t = edge_index_t, edge_weight_t
    ei_s, ew_s = edge_index_s, edge_weight_s
    inc = edge_index
    nt, ne = n0, e0
    deg = _sc_bincount(inc.reshape(-1), nt)

    for i, f in enumerate(filters):
        for j in range(channels[i]):
            mt_parts, mt_chunks = _messages(xs0, inc, nt)
            m_s = _gather_diff(xt0, inc, ne)
            xt_i = _msg_mm(xt0, mt_parts, mt_chunks, deg,
                           p["int%d%d_Wt" % (i, j)], "relu")
            xs_i = _msg_mm(xs0, [m_s], None, None,
                           p["int%d%d_Ws" % (i, j)], "relu")
            xt = _laguerre_bn(xt_i, ei_t, ew_t,
                              p["convt%d%d_W0" % (i, j)],
                              p["convt%d%d_W1" % (i, j)], nt)
            xs = _laguerre_bn(xs_i, ei_s, ew_s,
                              p["convs%d%d_W0" % (i, j)],
                              p["convs%d%d_W1" % (i, j)], ne)
            xt0 = jnp.concatenate([xt0, xt], axis=-1)
            xs0 = jnp.concatenate([xs0, xs], axis=-1)
        if i == 0:
            mt_parts, mt_chunks = _messages(xs0, inc, nt)
            m_s = _gather_diff(xt0, inc, ne)
            at = _msg_mm(xt0, mt_parts, mt_chunks, deg, p["att_Wt"], "attsig")
            as_ = _msg_mm(xs0, [m_s], None, None, p["att_Ws"], "attsig")
            cnt_t = _sc_bincount(pos_t, n1)
            cnt_s = _sc_bincount(pos_s, e1)
            xt0 = _scatter_mean(at, nt, pos_t, cnt_t, n1)
            xs0 = _scatter_mean(as_, ne, pos_s, cnt_s, e1)
            ei_t, ew_t = edge_index_t1, edge_weight_t1
            ei_s, ew_s = edge_index_s1, edge_weight_s1
            inc = edge_index1
            nt, ne = n1, e1
            deg = _sc_bincount(inc.reshape(-1), nt)

    cnt_nb = _sc_bincount(n_batch1, ngraph)
    cnt_sb = _sc_bincount(s_batch1, ngraph)
    g_s = _scatter_mean(xs, ne, s_batch1, cnt_sb, ngraph)
    g_t = _scatter_mean(xt, nt, n_batch1, cnt_nb, ngraph)
    xg = jnp.concatenate([g_s, g_t], axis=-1)

    wout = p["out_W"]
    bout = jnp.broadcast_to(p["out_b"][None, :], (8, wout.shape[1]))
    return _head_builder(wout.shape[0], wout.shape[1])(xg, wout, bout)


# speedup vs baseline: 2.9105x; 2.9105x over previous
import functools

import jax
import jax.numpy as jnp
from jax import lax
from jax.experimental import pallas as pl
from jax.experimental.pallas import tpu as pltpu
from jax.experimental.pallas import tpu_sc as plsc

F32 = jnp.float32
I32 = jnp.int32

_CH = 64
_NW = 32
_STRIDE = _NW * _CH
_ROW_PAD = 1024
_SPMEM_BUDGET = 4 * 1024 * 1024


def _ceil_to(x, m):
    return -(-x // m) * m


def _feat_chunks(d, n_pad):
    max_dc = min(512, (7_200_000 // (4 * n_pad + 4096)) // 16 * 16)
    nc = -(-d // max_dc)
    base = (d // nc) // 16 * 16
    sizes = [base] * (nc - 1) + [d - base * (nc - 1)]
    assert all(16 <= s <= max_dc and s % 16 == 0 for s in sizes), (d, n_pad, sizes)
    return tuple(sizes)


@functools.lru_cache(maxsize=None)
def _sc_scatter_builder(nx, dc, nnz_pad, n_pad, mode):
    nchunk = nnz_pad // (_NW * _CH)
    rows_per_tile = n_pad // 16
    n_copies = rows_per_tile // _CH
    ncol = dc // 16
    mesh = plsc.VectorSubcoreMesh(core_axis_name="c", subcore_axis_name="s")

    scratch = []
    if mode != "ones":
        scratch.append(pltpu.VMEM((_CH,), I32))
    scratch.append(pltpu.VMEM((_CH,), I32))
    if mode == "gs":
        scratch.append(pltpu.VMEM((_CH, 16), F32))
    scratch += [
        pltpu.VMEM((_CH, dc), F32),
        pltpu.VMEM_SHARED((n_pad, dc), F32),
        pltpu.SemaphoreType.DMA,
    ]

    def kern(*args):
        if mode == "gs":
            (x_hbm, gidx_hbm, dst_hbm, w_hbm, out_hbm,
             gidx_v, dst_v, w_v, rows_v, acc, sem) = args
        elif mode == "g":
            (x_hbm, gidx_hbm, dst_hbm, out_hbm,
             gidx_v, dst_v, rows_v, acc, sem) = args
        else:
            (dst_hbm, out_hbm, dst_v, rows_v, acc, sem) = args
        c = lax.axis_index("c")
        s = lax.axis_index("s")
        wid = s * 2 + c
        t0 = s * rows_per_tile

        def fill(val):
            vec = jnp.full((16,), val, F32)

            def row(r, carry):
                for k in range(ncol):
                    rows_v[r, pl.ds(16 * k, 16)] = vec
                return carry

            lax.fori_loop(0, _CH, row, 0)

        fill(0.0)
        for j in range(n_copies):
            pltpu.sync_copy(rows_v, acc.at[pl.ds(t0 + j * _CH, _CH)])
        plsc.subcore_barrier()
        if mode == "ones":
            fill(1.0)

        def chunk(ci, carry):
            base = (wid * nchunk + ci) * _CH
            pltpu.sync_copy(dst_hbm.at[pl.ds(base, _CH)], dst_v)
            if mode != "ones":
                pltpu.sync_copy(gidx_hbm.at[pl.ds(base, _CH)], gidx_v)
                pltpu.async_copy(x_hbm.at[gidx_v], rows_v, sem).wait()
            if mode == "gs":
                pltpu.sync_copy(w_hbm.at[pl.ds(base, _CH)], w_v)

                def sgrp(g, carry2):
                    for r8 in range(8):
                        r = g * 8 + r8
                        ws = w_v[r, pl.ds(0, 16)]
                        for k in range(ncol):
                            sl = pl.ds(16 * k, 16)
                            rows_v[r, sl] = rows_v[r, sl] * ws
                    return carry2

                lax.fori_loop(0, _CH // 8, sgrp, 0)
            pltpu.sync_copy(rows_v, acc.at[dst_v], add=True)
            return carry

        lax.fori_loop(0, nchunk, chunk, 0)
        plsc.subcore_barrier()
        for j in range(n_copies):
            sl = pl.ds(t0 + j * _CH, _CH)
            pltpu.sync_copy(acc.at[sl], out_hbm.at[c, sl])

    return pl.kernel(
        kern,
        out_type=jax.ShapeDtypeStruct((2, n_pad, dc), F32),
        mesh=mesh,
        scratch_types=scratch,
        compiler_params=pltpu.CompilerParams(use_tc_tiling_on_sc=False),
    )


@functools.lru_cache(maxsize=None)
def _sc_diff_builder(nx, dc, e_pad):
    nchunk = e_pad // (_NW * _CH)
    ncol = dc // 16
    mesh = plsc.VectorSubcoreMesh(core_axis_name="c", subcore_axis_name="s")

    def kern(x_hbm, src_hbm, dst_hbm, out_hbm, si_v, di_v, ra_v, rb_v,
             sem_a, sem_b):
        c = lax.axis_index("c")
        s = lax.axis_index("s")
        wid = s * 2 + c

        def chunk(ci, carry):
            base = (wid * nchunk + ci) * _CH
            pltpu.sync_copy(dst_hbm.at[pl.ds(base, _CH)], di_v)
            pltpu.sync_copy(src_hbm.at[pl.ds(base, _CH)], si_v)
            ca = pltpu.async_copy(x_hbm.at[di_v], ra_v, sem_a)
            cb = pltpu.async_copy(x_hbm.at[si_v], rb_v, sem_b)
            ca.wait()
            cb.wait()

            def grp(g, carry2):
                for r8 in range(8):
                    r = g * 8 + r8
                    for k in range(ncol):
                        sl = pl.ds(16 * k, 16)
                        ra_v[r, sl] = ra_v[r, sl] - rb_v[r, sl]
                return carry2

            lax.fori_loop(0, _CH // 8, grp, 0)
            pltpu.sync_copy(ra_v, out_hbm.at[pl.ds(base, _CH)])
            return carry

        lax.fori_loop(0, nchunk, chunk, 0)

    return pl.kernel(
        kern,
        out_type=jax.ShapeDtypeStruct((e_pad, dc), F32),
        mesh=mesh,
        scratch_types=[
            pltpu.VMEM((_CH,), I32), pltpu.VMEM((_CH,), I32),
            pltpu.VMEM((_CH, dc), F32), pltpu.VMEM((_CH, dc), F32),
            pltpu.SemaphoreType.DMA, pltpu.SemaphoreType.DMA,
        ],
        compiler_params=pltpu.CompilerParams(use_tc_tiling_on_sc=False),
    )


def _pad_entries(arr, nnz_pad, value):
    n = arr.shape[0]
    if n == nnz_pad:
        return arr
    return jnp.concatenate([arr, jnp.full((nnz_pad - n,), value, arr.dtype)])


def _sc_scatter(x, gidx, dst, w, n_out):
    nnz = dst.shape[0]
    nnz_pad = _ceil_to(nnz, _STRIDE)
    n_pad = _ceil_to(n_out + 1, _ROW_PAD)
    gidx_p = _pad_entries(gidx, nnz_pad, 0)
    dst_p = _pad_entries(dst, nnz_pad, n_out)
    w_p = None
    if w is not None:
        w_p = jnp.broadcast_to(_pad_entries(w, nnz_pad, 0.0)[:, None],
                               (nnz_pad, 16))
    d = x.shape[1]
    parts = []
    c0 = 0
    chunks = _feat_chunks(d, n_pad)
    for dc in chunks:
        xc = lax.slice_in_dim(x, c0, c0 + dc, axis=1)
        if w is None:
            fn = _sc_scatter_builder(x.shape[0], dc, nnz_pad, n_pad, "g")
            parts.append(fn(xc, gidx_p, dst_p))
        else:
            fn = _sc_scatter_builder(x.shape[0], dc, nnz_pad, n_pad, "gs")
            parts.append(fn(xc, gidx_p, dst_p, w_p))
        c0 += dc
    return parts, chunks


def _sc_bincount(idx, n_out):
    nnz = idx.shape[0]
    nnz_pad = _ceil_to(nnz, _STRIDE)
    n_pad = _ceil_to(n_out + 1, _ROW_PAD)
    dst_p = _pad_entries(idx, nnz_pad, n_out)
    fn = _sc_scatter_builder(0, 16, nnz_pad, n_pad, "ones")
    return fn(dst_p)


def _row_block(n_pad, d_tot):
    br = 2048 if n_pad % 2048 == 0 else n_pad
    if d_tot >= 704 and br > 1024:
        br = 1024
    return br


@functools.lru_cache(maxsize=None)
def _mm_stats_builder(n_pad, n_true, k, f):
    br = _row_block(n_pad, k)
    grid = n_pad // br

    def body(x_ref, w_ref, y_ref, st_ref):
        y = jnp.dot(x_ref[...], w_ref[...], preferred_element_type=F32)
        y_ref[...] = y
        i = pl.program_id(0)

        @pl.when(i == 0)
        def _():
            st_ref[...] = jnp.zeros_like(st_ref)

        row = i * br + lax.broadcasted_iota(I32, (br, 1), 0)
        ym = jnp.where(row < n_true, y, 0.0)
        st_ref[0:1, :] = st_ref[0:1, :] + jnp.sum(ym, axis=0, keepdims=True)
        st_ref[1:2, :] = st_ref[1:2, :] + jnp.sum(ym * ym, axis=0, keepdims=True)

    return pl.pallas_call(
        body,
        grid=(grid,),
        in_specs=[pl.BlockSpec((br, k), lambda i: (i, 0)),
                  pl.BlockSpec((k, f), lambda i: (0, 0))],
        out_specs=[pl.BlockSpec((br, f), lambda i: (i, 0)),
                   pl.BlockSpec((8, f), lambda i: (0, 0))],
        out_shape=[jax.ShapeDtypeStruct((n_pad, f), F32),
                   jax.ShapeDtypeStruct((8, f), F32)],
    )


@functools.lru_cache(maxsize=None)
def _laguerre_builder(n_pad, n_true, d, f, chunks):
    br = _row_block(n_pad, d)
    grid = n_pad // br
    nchunks = len(chunks)

    def body(*refs):
        x_ref = refs[0]
        p_refs = refs[1:1 + nchunks]
        w0_ref, w1_ref, y_ref, st_ref = refs[1 + nchunks:]
        x = x_ref[...]
        lx = jnp.concatenate([p[0] + p[1] for p in p_refs], axis=-1) \
            if nchunks > 1 else (p_refs[0][0] + p_refs[0][1])
        y = (jnp.dot(x, w0_ref[...], preferred_element_type=F32)
             + jnp.dot(x - lx, w1_ref[...], preferred_element_type=F32))
        y_ref[...] = y
        i = pl.program_id(0)

        @pl.when(i == 0)
        def _():
            st_ref[...] = jnp.zeros_like(st_ref)

        row = i * br + lax.broadcasted_iota(I32, (br, 1), 0)
        ym = jnp.where(row < n_true, y, 0.0)
        st_ref[0:1, :] = st_ref[0:1, :] + jnp.sum(ym, axis=0, keepdims=True)
        st_ref[1:2, :] = st_ref[1:2, :] + jnp.sum(ym * ym, axis=0, keepdims=True)

    in_specs = [pl.BlockSpec((br, d), lambda i: (i, 0))]
    for dc in chunks:
        in_specs.append(pl.BlockSpec((2, br, dc), lambda i: (0, i, 0)))
    in_specs += [pl.BlockSpec((d, f), lambda i: (0, 0)),
                 pl.BlockSpec((d, f), lambda i: (0, 0))]
    return pl.pallas_call(
        body,
        grid=(grid,),
        in_specs=in_specs,
        out_specs=[pl.BlockSpec((br, f), lambda i: (i, 0)),
                   pl.BlockSpec((8, f), lambda i: (0, 0))],
        out_shape=[jax.ShapeDtypeStruct((n_pad, f), F32),
                   jax.ShapeDtypeStruct((8, f), F32)],
    )


@functools.lru_cache(maxsize=None)
def _msg_mm_builder(n_pad, d, f, chunks, use_counts, act):
    br = _row_block(n_pad, d)
    grid = n_pad // br
    nchunks = 1 if chunks is None else len(chunks)

    def body(*refs):
        x_ref = refs[0]
        p_refs = refs[1:1 + nchunks]
        rest = refs[1 + nchunks:]
        if use_counts:
            c_ref, w_ref, o_ref = rest
        else:
            w_ref, o_ref = rest
        if chunks is None:
            m = p_refs[0][...]
        else:
            m = jnp.concatenate([p[0] + p[1] for p in p_refs], axis=-1) \
                if nchunks > 1 else (p_refs[0][0] + p_refs[0][1])
        if use_counts:
            cnt = c_ref[0, :, 0:1] + c_ref[1, :, 0:1]
            m = m / (cnt + 1e-6)
        x = x_ref[...]
        y = jnp.dot(x + m, w_ref[...], preferred_element_type=F32)
        if act == "relu":
            o_ref[...] = jnp.maximum(y, 0.0)
        else:
            o_ref[...] = x * jax.nn.sigmoid(y)

    in_specs = [pl.BlockSpec((br, d), lambda i: (i, 0))]
    if chunks is None:
        in_specs.append(pl.BlockSpec((br, d), lambda i: (i, 0)))
    else:
        for dc in chunks:
            in_specs.append(pl.BlockSpec((2, br, dc), lambda i: (0, i, 0)))
    if use_counts:
        in_specs.append(pl.BlockSpec((2, br, 16), lambda i: (0, i, 0)))
    in_specs.append(pl.BlockSpec((d, f), lambda i: (0, 0)))
    return pl.pallas_call(
        body,
        grid=(grid,),
        in_specs=in_specs,
        out_specs=pl.BlockSpec((br, f), lambda i: (i, 0)),
        out_shape=jax.ShapeDtypeStruct((n_pad, f), F32),
    )


@functools.lru_cache(maxsize=None)
def _bn_relu_builder(n_pad, n_true, f):
    br = _row_block(n_pad, f)
    grid = n_pad // br
    inv_n = 1.0 / n_true

    def body(y_ref, st_ref, o_ref):
        mu = st_ref[0:1, :] * inv_n
        var = st_ref[1:2, :] * inv_n - mu * mu
        y = jnp.maximum((y_ref[...] - mu) * lax.rsqrt(var + 1e-5), 0.0)
        row = pl.program_id(0) * br + lax.broadcasted_iota(I32, (br, 1), 0)
        o_ref[...] = jnp.where(row < n_true, y, 0.0)

    return pl.pallas_call(
        body,
        grid=(grid,),
        in_specs=[pl.BlockSpec((br, f), lambda i: (i, 0)),
                  pl.BlockSpec((8, f), lambda i: (0, 0))],
        out_specs=pl.BlockSpec((br, f), lambda i: (i, 0)),
        out_shape=jax.ShapeDtypeStruct((n_pad, f), F32),
    )


@functools.lru_cache(maxsize=None)
def _mean_combine_builder(n_pad, dc):
    br = 2048 if n_pad % 2048 == 0 else n_pad
    grid = n_pad // br

    def body(p_ref, c_ref, o_ref):
        cnt = c_ref[0, :, 0:1] + c_ref[1, :, 0:1]
        o_ref[...] = (p_ref[0] + p_ref[1]) / jnp.maximum(cnt, 1.0)

    return pl.pallas_call(
        body,
        grid=(grid,),
        in_specs=[pl.BlockSpec((2, br, dc), lambda i: (0, i, 0)),
                  pl.BlockSpec((2, br, 16), lambda i: (0, i, 0))],
        out_specs=pl.BlockSpec((br, dc), lambda i: (i, 0)),
        out_shape=jax.ShapeDtypeStruct((n_pad, dc), F32),
    )


@functools.lru_cache(maxsize=None)
def _head_builder(k, f):
    def body(x_ref, w_ref, b_ref, o_ref):
        o_ref[...] = (jnp.dot(x_ref[...], w_ref[...], preferred_element_type=F32)
                      + b_ref[0:1, :])

    return pl.pallas_call(
        body,
        grid=(1,),
        in_specs=[pl.BlockSpec((64, k), lambda i: (0, 0)),
                  pl.BlockSpec((k, f), lambda i: (0, 0)),
                  pl.BlockSpec((8, f), lambda i: (0, 0))],
        out_specs=pl.BlockSpec((64, f), lambda i: (0, 0)),
        out_shape=jax.ShapeDtypeStruct((64, f), F32),
    )


def _bn_relu(y, st, n_true):
    return _bn_relu_builder(y.shape[0], n_true, y.shape[1])(y, st)


def _messages(x_table, inc, n_out):
    src, dst = inc[0], inc[1]
    e = src.shape[0]
    ar = jnp.arange(e, dtype=I32)
    ones = jnp.ones((e,), F32)
    gidx = jnp.concatenate([ar, ar])
    dsts = jnp.concatenate([src, dst])
    w = jnp.concatenate([-ones, ones])
    return _sc_scatter(x_table, gidx, dsts, w, n_out)


def _gather_diff(x_table, inc, n_out):
    src, dst = inc[0], inc[1]
    e = src.shape[0]
    e_pad = _ceil_to(e, _STRIDE)
    src_p = _pad_entries(src, e_pad, 0)
    dst_p = _pad_entries(dst, e_pad, 0)
    d = x_table.shape[1]
    nc = -(-d // 512)
    base = (d // nc) // 16 * 16
    sizes = [base] * (nc - 1) + [d - base * (nc - 1)]
    outs = []
    c0 = 0
    for dc in sizes:
        xc = lax.slice_in_dim(x_table, c0, c0 + dc, axis=1)
        outs.append(_sc_diff_builder(x_table.shape[0], dc, e_pad)(
            xc, src_p, dst_p))
        c0 += dc
    return jnp.concatenate(outs, axis=-1) if len(outs) > 1 else outs[0]


def _msg_mm(x, parts, chunks, counts, w, act):
    n_pad, d = x.shape
    fn = _msg_mm_builder(n_pad, d, w.shape[1], chunks, counts is not None, act)
    args = [x] + list(parts)
    if counts is not None:
        args.append(counts)
    args.append(w)
    return fn(*args)


def _laguerre_bn(x, ei, ew, w0, w1, n_true):
    n_pad, d = x.shape
    parts, chunks = _sc_scatter(x, ei[0], ei[1], ew, n_true)
    y, st = _laguerre_builder(n_pad, n_true, d, w0.shape[1], chunks)(
        x, *parts, w0, w1)
    return _bn_relu(y, st, n_true)


def _scatter_mean(x_table, n_rows, idx, counts, n_out):
    ar = jnp.arange(n_rows, dtype=I32)
    parts, chunks = _sc_scatter(x_table, ar, idx, None, n_out)
    n_pad = parts[0].shape[1]
    outs = [_mean_combine_builder(n_pad, dc)(p, counts)
            for p, dc in zip(parts, chunks)]
    return jnp.concatenate(outs, axis=-1) if len(outs) > 1 else outs[0]


def kernel(x_t, x_s, edge_weight_t, edge_weight_s, edge_weight_t1,
           edge_weight_s1, params, edge_index_t, edge_index_s, edge_index,
           edge_index_t1, edge_index_s1, edge_index1, pos_t, pos_s,
           n_batch1, s_batch1):
    p = params
    filters = [64, 128, 256, 512]
    channels = [2, 2, 2, 2]
    n0 = x_t.shape[0]
    e0 = x_s.shape[0]
    n1 = edge_index_t1.shape[1] // 3 * 0 + 2000
    e1 = 2000
    ngraph = 64
    n0_pad = _ceil_to(n0 + 1, _ROW_PAD)
    e0_pad = _ceil_to(e0 + 1, _ROW_PAD)

    kt = _ceil_to(x_t.shape[1], 128)
    ks = _ceil_to(x_s.shape[1], 128)
    xtp = jnp.pad(x_t, ((0, n0_pad - n0), (0, kt - x_t.shape[1])))
    xsp = jnp.pad(x_s, ((0, e0_pad - e0), (0, ks - x_s.shape[1])))
    wt0 = jnp.pad(p["init_Wt"], ((0, kt - p["init_Wt"].shape[0]), (0, 0)))
    ws0 = jnp.pad(p["init_Ws"], ((0, ks - p["init_Ws"].shape[0]), (0, 0)))
    y, st = _mm_stats_builder(n0_pad, n0, kt, 64)(xtp, wt0)
    xt = _bn_relu(y, st, n0)
    y, st = _mm_stats_builder(e0_pad, e0, ks, 64)(xsp, ws0)
    xs = _bn_relu(y, st, e0)

    xt0, xs0 = xt, xs
    ei_t, ew_t = edge_index_t, edge_weight_t
    ei_s, ew_s = edge_index_s, edge_weight_s
    inc = edge_index
    nt, ne = n0, e0
    deg = _sc_bincount(inc.reshape(-1), nt)

    for i, f in enumerate(filters):
        for j in range(channels[i]):
            mt_parts, mt_chunks = _messages(xs0, inc, nt)
            m_s = _gather_diff(xt0, inc, ne)
            xt_i = _msg_mm(xt0, mt_parts, mt_chunks, deg,
                           p["int%d%d_Wt" % (i, j)], "relu")
            xs_i = _msg_mm(xs0, [m_s], None, None,
                           p["int%d%d_Ws" % (i, j)], "relu")
            xt = _laguerre_bn(xt_i, ei_t, ew_t,
                              p["convt%d%d_W0" % (i, j)],
                              p["convt%d%d_W1" % (i, j)], nt)
            xs = _laguerre_bn(xs_i, ei_s, ew_s,
                              p["convs%d%d_W0" % (i, j)],
                              p["convs%d%d_W1" % (i, j)], ne)
            xt0 = jnp.concatenate([xt0, xt], axis=-1)
            xs0 = jnp.concatenate([xs0, xs], axis=-1)
        if i == 0:
            mt_parts, mt_chunks = _messages(xs0, inc, nt)
            m_s = _gather_diff(xt0, inc, ne)
            at = _msg_mm(xt0, mt_parts, mt_chunks, deg, p["att_Wt"], "attsig")
            as_ = _msg_mm(xs0, [m_s], None, None, p["att_Ws"], "attsig")
            cnt_t = _sc_bincount(pos_t, n1)
            cnt_s = _sc_bincount(pos_s, e1)
            xt0 = _scatter_mean(at, nt, pos_t, cnt_t, n1)
            xs0 = _scatter_mean(as_, ne, pos_s, cnt_s, e1)
            ei_t, ew_t = edge_index_t1, edge_weight_t1
            ei_s, ew_s = edge_index_s1, edge_weight_s1
            inc = edge_index1
            nt, ne = n1, e1
            deg = _sc_bincount(inc.reshape(-1), nt)

    cnt_nb = _sc_bincount(n_batch1, ngraph)
    cnt_sb = _sc_bincount(s_batch1, ngraph)
    g_s = _scatter_mean(xs, ne, s_batch1, cnt_sb, ngraph)
    g_t = _scatter_mean(xt, nt, n_batch1, cnt_nb, ngraph)
    xg = jnp.concatenate([g_s, g_t], axis=-1)

    wout = p["out_W"]
    bout = jnp.broadcast_to(p["out_b"][None, :], (8, wout.shape[1]))
    return _head_builder(wout.shape[0], wout.shape[1])(xg, wout, bout)


# speedup vs baseline: 3.0109x; 1.0345x over previous
import functools

import jax
import jax.numpy as jnp
from jax import lax
from jax.experimental import pallas as pl
from jax.experimental.pallas import tpu as pltpu
from jax.experimental.pallas import tpu_sc as plsc

F32 = jnp.float32
I32 = jnp.int32

_CH = 64
_NW = 32
_STRIDE = _NW * _CH
_ROW_PAD = 1024
_SPMEM_BUDGET = 4 * 1024 * 1024


def _ceil_to(x, m):
    return -(-x // m) * m


def _feat_chunks(d, n_pad):
    max_dc = min(512, (7_200_000 // (4 * n_pad + 4096)) // 16 * 16)
    nc = -(-d // max_dc)
    base = (d // nc) // 16 * 16
    sizes = [base] * (nc - 1) + [d - base * (nc - 1)]
    assert all(16 <= s <= max_dc and s % 16 == 0 for s in sizes), (d, n_pad, sizes)
    return tuple(sizes)


@functools.lru_cache(maxsize=None)
def _sc_scatter_builder(nx, dc, nnz_pad, n_pad, mode):
    nchunk = nnz_pad // (_NW * _CH)
    rows_per_tile = n_pad // 16
    n_copies = rows_per_tile // _CH
    ncol = dc // 16
    mesh = plsc.VectorSubcoreMesh(core_axis_name="c", subcore_axis_name="s")

    scratch = []
    if mode != "ones":
        scratch.append(pltpu.VMEM((_CH,), I32))
    scratch.append(pltpu.VMEM((_CH,), I32))
    if mode == "gs":
        scratch.append(pltpu.VMEM((_CH, 16), F32))
    scratch += [
        pltpu.VMEM((_CH, dc), F32),
        pltpu.VMEM_SHARED((n_pad, dc), F32),
        pltpu.SemaphoreType.DMA,
    ]

    def kern(*args):
        if mode == "gs":
            (x_hbm, gidx_hbm, dst_hbm, w_hbm, out_hbm,
             gidx_v, dst_v, w_v, rows_v, acc, sem) = args
        elif mode == "g":
            (x_hbm, gidx_hbm, dst_hbm, out_hbm,
             gidx_v, dst_v, rows_v, acc, sem) = args
        else:
            (dst_hbm, out_hbm, dst_v, rows_v, acc, sem) = args
        c = lax.axis_index("c")
        s = lax.axis_index("s")
        wid = s * 2 + c
        t0 = s * rows_per_tile

        def fill(val):
            vec = jnp.full((16,), val, F32)

            def row(r, carry):
                for k in range(ncol):
                    rows_v[r, pl.ds(16 * k, 16)] = vec
                return carry

            lax.fori_loop(0, _CH, row, 0)

        fill(0.0)
        for j in range(n_copies):
            pltpu.sync_copy(rows_v, acc.at[pl.ds(t0 + j * _CH, _CH)])
        plsc.subcore_barrier()
        if mode == "ones":
            fill(1.0)

        def chunk(ci, carry):
            base = (wid * nchunk + ci) * _CH
            pltpu.sync_copy(dst_hbm.at[pl.ds(base, _CH)], dst_v)
            if mode != "ones":
                pltpu.sync_copy(gidx_hbm.at[pl.ds(base, _CH)], gidx_v)
                pltpu.async_copy(x_hbm.at[gidx_v], rows_v, sem).wait()
            if mode == "gs":
                pltpu.sync_copy(w_hbm.at[pl.ds(base, _CH)], w_v)

                def sgrp(g, carry2):
                    for r8 in range(8):
                        r = g * 8 + r8
                        ws = w_v[r, pl.ds(0, 16)]
                        for k in range(ncol):
                            sl = pl.ds(16 * k, 16)
                            rows_v[r, sl] = rows_v[r, sl] * ws
                    return carry2

                lax.fori_loop(0, _CH // 8, sgrp, 0)
            pltpu.sync_copy(rows_v, acc.at[dst_v], add=True)
            return carry

        lax.fori_loop(0, nchunk, chunk, 0)
        plsc.subcore_barrier()
        for j in range(n_copies):
            sl = pl.ds(t0 + j * _CH, _CH)
            pltpu.sync_copy(acc.at[sl], out_hbm.at[c, sl])

    return pl.kernel(
        kern,
        out_type=jax.ShapeDtypeStruct((2, n_pad, dc), F32),
        mesh=mesh,
        scratch_types=scratch,
        compiler_params=pltpu.CompilerParams(use_tc_tiling_on_sc=False),
    )


@functools.lru_cache(maxsize=None)
def _sc_diff_builder(nx, dc, e_pad):
    nchunk = e_pad // (_NW * _CH)
    ncol = dc // 16
    mesh = plsc.VectorSubcoreMesh(core_axis_name="c", subcore_axis_name="s")

    def kern(x_hbm, src_hbm, dst_hbm, out_hbm, si_v, di_v, ra_v, rb_v,
             sem_a, sem_b):
        c = lax.axis_index("c")
        s = lax.axis_index("s")
        wid = s * 2 + c

        def chunk(ci, carry):
            base = (wid * nchunk + ci) * _CH
            pltpu.sync_copy(dst_hbm.at[pl.ds(base, _CH)], di_v)
            pltpu.sync_copy(src_hbm.at[pl.ds(base, _CH)], si_v)
            ca = pltpu.async_copy(x_hbm.at[di_v], ra_v, sem_a)
            cb = pltpu.async_copy(x_hbm.at[si_v], rb_v, sem_b)
            ca.wait()
            cb.wait()

            def grp(g, carry2):
                for r8 in range(8):
                    r = g * 8 + r8
                    for k in range(ncol):
                        sl = pl.ds(16 * k, 16)
                        ra_v[r, sl] = ra_v[r, sl] - rb_v[r, sl]
                return carry2

            lax.fori_loop(0, _CH // 8, grp, 0)
            pltpu.sync_copy(ra_v, out_hbm.at[pl.ds(base, _CH)])
            return carry

        lax.fori_loop(0, nchunk, chunk, 0)

    return pl.kernel(
        kern,
        out_type=jax.ShapeDtypeStruct((e_pad, dc), F32),
        mesh=mesh,
        scratch_types=[
            pltpu.VMEM((_CH,), I32), pltpu.VMEM((_CH,), I32),
            pltpu.VMEM((_CH, dc), F32), pltpu.VMEM((_CH, dc), F32),
            pltpu.SemaphoreType.DMA, pltpu.SemaphoreType.DMA,
        ],
        compiler_params=pltpu.CompilerParams(use_tc_tiling_on_sc=False),
    )


def _pad_entries(arr, nnz_pad, value):
    n = arr.shape[0]
    if n == nnz_pad:
        return arr
    return jnp.concatenate([arr, jnp.full((nnz_pad - n,), value, arr.dtype)])


def _sc_scatter(x, gidx, dst, w, n_out):
    nnz = dst.shape[0]
    nnz_pad = _ceil_to(nnz, _STRIDE)
    n_pad = _ceil_to(n_out + 1, _ROW_PAD)
    gidx_p = _pad_entries(gidx, nnz_pad, 0)
    dst_p = _pad_entries(dst, nnz_pad, n_out)
    w_p = None
    if w is not None:
        w_p = jnp.broadcast_to(_pad_entries(w, nnz_pad, 0.0)[:, None],
                               (nnz_pad, 16))
    d = x.shape[1]
    parts = []
    c0 = 0
    chunks = _feat_chunks(d, n_pad)
    for dc in chunks:
        xc = lax.slice_in_dim(x, c0, c0 + dc, axis=1)
        if w is None:
            fn = _sc_scatter_builder(x.shape[0], dc, nnz_pad, n_pad, "g")
            parts.append(fn(xc, gidx_p, dst_p))
        else:
            fn = _sc_scatter_builder(x.shape[0], dc, nnz_pad, n_pad, "gs")
            parts.append(fn(xc, gidx_p, dst_p, w_p))
        c0 += dc
    return parts, chunks


def _sc_bincount(idx, n_out):
    nnz = idx.shape[0]
    nnz_pad = _ceil_to(nnz, _STRIDE)
    n_pad = _ceil_to(n_out + 1, _ROW_PAD)
    dst_p = _pad_entries(idx, nnz_pad, n_out)
    fn = _sc_scatter_builder(0, 16, nnz_pad, n_pad, "ones")
    return fn(dst_p)


def _row_block(n_pad, d_tot):
    br = 2048 if n_pad % 2048 == 0 else n_pad
    if d_tot >= 704 and br > 1024:
        br = 1024
    return br


@functools.lru_cache(maxsize=None)
def _mm_stats_builder(n_pad, n_true, k, f):
    br = _row_block(n_pad, k)
    grid = n_pad // br

    def body(x_ref, w_ref, y_ref, st_ref):
        y = jnp.dot(x_ref[...], w_ref[...], preferred_element_type=F32)
        y_ref[...] = y
        i = pl.program_id(0)

        @pl.when(i == 0)
        def _():
            st_ref[...] = jnp.zeros_like(st_ref)

        row = i * br + lax.broadcasted_iota(I32, (br, 1), 0)
        ym = jnp.where(row < n_true, y, 0.0)
        st_ref[0:1, :] = st_ref[0:1, :] + jnp.sum(ym, axis=0, keepdims=True)
        st_ref[1:2, :] = st_ref[1:2, :] + jnp.sum(ym * ym, axis=0, keepdims=True)

    return pl.pallas_call(
        body,
        grid=(grid,),
        in_specs=[pl.BlockSpec((br, k), lambda i: (i, 0)),
                  pl.BlockSpec((k, f), lambda i: (0, 0))],
        out_specs=[pl.BlockSpec((br, f), lambda i: (i, 0)),
                   pl.BlockSpec((8, f), lambda i: (0, 0))],
        out_shape=[jax.ShapeDtypeStruct((n_pad, f), F32),
                   jax.ShapeDtypeStruct((8, f), F32)],
    )


@functools.lru_cache(maxsize=None)
def _laguerre_builder(n_pad, n_true, d, f, chunks):
    br = _row_block(n_pad, d)
    grid = n_pad // br
    nchunks = len(chunks)

    def body(*refs):
        x_ref = refs[0]
        p_refs = refs[1:1 + nchunks]
        w0_ref, w1_ref, y_ref, st_ref = refs[1 + nchunks:]
        x = x_ref[...]
        lx = jnp.concatenate([p[0] + p[1] for p in p_refs], axis=-1) \
            if nchunks > 1 else (p_refs[0][0] + p_refs[0][1])
        y = (jnp.dot(x, w0_ref[...], preferred_element_type=F32)
             + jnp.dot(x - lx, w1_ref[...], preferred_element_type=F32))
        y_ref[...] = y
        i = pl.program_id(0)

        @pl.when(i == 0)
        def _():
            st_ref[...] = jnp.zeros_like(st_ref)

        row = i * br + lax.broadcasted_iota(I32, (br, 1), 0)
        ym = jnp.where(row < n_true, y, 0.0)
        st_ref[0:1, :] = st_ref[0:1, :] + jnp.sum(ym, axis=0, keepdims=True)
        st_ref[1:2, :] = st_ref[1:2, :] + jnp.sum(ym * ym, axis=0, keepdims=True)

    in_specs = [pl.BlockSpec((br, d), lambda i: (i, 0))]
    for dc in chunks:
        in_specs.append(pl.BlockSpec((2, br, dc), lambda i: (0, i, 0)))
    in_specs += [pl.BlockSpec((d, f), lambda i: (0, 0)),
                 pl.BlockSpec((d, f), lambda i: (0, 0))]
    return pl.pallas_call(
        body,
        grid=(grid,),
        in_specs=in_specs,
        out_specs=[pl.BlockSpec((br, f), lambda i: (i, 0)),
                   pl.BlockSpec((8, f), lambda i: (0, 0))],
        out_shape=[jax.ShapeDtypeStruct((n_pad, f), F32),
                   jax.ShapeDtypeStruct((8, f), F32)],
    )


@functools.lru_cache(maxsize=None)
def _mm_builder(n_pad, k, f):
    br = _row_block(n_pad, k)
    grid = n_pad // br

    def body(x_ref, w_ref, y_ref):
        y_ref[...] = jnp.dot(x_ref[...], w_ref[...], preferred_element_type=F32)

    return pl.pallas_call(
        body,
        grid=(grid,),
        in_specs=[pl.BlockSpec((br, k), lambda i: (i, 0)),
                  pl.BlockSpec((k, f), lambda i: (0, 0))],
        out_specs=pl.BlockSpec((br, f), lambda i: (i, 0)),
        out_shape=jax.ShapeDtypeStruct((n_pad, f), F32),
    )


@functools.lru_cache(maxsize=None)
def _msg_post_builder(n_pad, d, f, chunks, act):
    br = _row_block(n_pad, max(d, f))
    grid = n_pad // br
    nchunks = 0 if chunks is None else len(chunks)

    def body(*refs):
        x_ref = refs[0]
        if chunks is None:
            m = refs[1][...]
            w_ref, o_ref = refs[2:]
        else:
            pos = refs[1:1 + nchunks]
            neg = refs[1 + nchunks:1 + 2 * nchunks]
            c_ref, w_ref, o_ref = refs[1 + 2 * nchunks:]
            ms = [pp[0] + pp[1] - qq[0] - qq[1] for pp, qq in zip(pos, neg)]
            m = jnp.concatenate(ms, axis=-1) if nchunks > 1 else ms[0]
            cnt = c_ref[0, :, 0:1] + c_ref[1, :, 0:1]
            m = m / (cnt + 1e-6)
        x = x_ref[...]
        y = jnp.dot(x, w_ref[...], preferred_element_type=F32) + m
        if act == "relu":
            o_ref[...] = jnp.maximum(y, 0.0)
        else:
            o_ref[...] = x * jax.nn.sigmoid(y)

    in_specs = [pl.BlockSpec((br, d), lambda i: (i, 0))]
    if chunks is None:
        in_specs.append(pl.BlockSpec((br, f), lambda i: (i, 0)))
    else:
        for dc in chunks + chunks:
            in_specs.append(pl.BlockSpec((2, br, dc), lambda i: (0, i, 0)))
        in_specs.append(pl.BlockSpec((2, br, 16), lambda i: (0, i, 0)))
    in_specs.append(pl.BlockSpec((d, f), lambda i: (0, 0)))
    return pl.pallas_call(
        body,
        grid=(grid,),
        in_specs=in_specs,
        out_specs=pl.BlockSpec((br, f), lambda i: (i, 0)),
        out_shape=jax.ShapeDtypeStruct((n_pad, f), F32),
    )


@functools.lru_cache(maxsize=None)
def _bn_relu_builder(n_pad, n_true, f):
    br = _row_block(n_pad, f)
    grid = n_pad // br
    inv_n = 1.0 / n_true

    def body(y_ref, st_ref, o_ref):
        mu = st_ref[0:1, :] * inv_n
        var = st_ref[1:2, :] * inv_n - mu * mu
        y = jnp.maximum((y_ref[...] - mu) * lax.rsqrt(var + 1e-5), 0.0)
        row = pl.program_id(0) * br + lax.broadcasted_iota(I32, (br, 1), 0)
        o_ref[...] = jnp.where(row < n_true, y, 0.0)

    return pl.pallas_call(
        body,
        grid=(grid,),
        in_specs=[pl.BlockSpec((br, f), lambda i: (i, 0)),
                  pl.BlockSpec((8, f), lambda i: (0, 0))],
        out_specs=pl.BlockSpec((br, f), lambda i: (i, 0)),
        out_shape=jax.ShapeDtypeStruct((n_pad, f), F32),
    )


@functools.lru_cache(maxsize=None)
def _mean_combine_builder(n_pad, dc):
    br = 2048 if n_pad % 2048 == 0 else n_pad
    grid = n_pad // br

    def body(p_ref, c_ref, o_ref):
        cnt = c_ref[0, :, 0:1] + c_ref[1, :, 0:1]
        o_ref[...] = (p_ref[0] + p_ref[1]) / jnp.maximum(cnt, 1.0)

    return pl.pallas_call(
        body,
        grid=(grid,),
        in_specs=[pl.BlockSpec((2, br, dc), lambda i: (0, i, 0)),
                  pl.BlockSpec((2, br, 16), lambda i: (0, i, 0))],
        out_specs=pl.BlockSpec((br, dc), lambda i: (i, 0)),
        out_shape=jax.ShapeDtypeStruct((n_pad, dc), F32),
    )


@functools.lru_cache(maxsize=None)
def _head_builder(k, f):
    def body(x_ref, w_ref, b_ref, o_ref):
        o_ref[...] = (jnp.dot(x_ref[...], w_ref[...], preferred_element_type=F32)
                      + b_ref[0:1, :])

    return pl.pallas_call(
        body,
        grid=(1,),
        in_specs=[pl.BlockSpec((64, k), lambda i: (0, 0)),
                  pl.BlockSpec((k, f), lambda i: (0, 0)),
                  pl.BlockSpec((8, f), lambda i: (0, 0))],
        out_specs=pl.BlockSpec((64, f), lambda i: (0, 0)),
        out_shape=jax.ShapeDtypeStruct((64, f), F32),
    )


def _bn_relu(y, st, n_true):
    return _bn_relu_builder(y.shape[0], n_true, y.shape[1])(y, st)


def _messages_z(z, inc, n_out):
    e = inc.shape[1]
    ar = jnp.arange(e, dtype=I32)
    pos, chunks = _sc_scatter(z, ar, inc[1], None, n_out)
    neg, _ = _sc_scatter(z, ar, inc[0], None, n_out)
    return pos, neg, chunks


def _gather_diff(x_table, inc, n_out):
    src, dst = inc[0], inc[1]
    e = src.shape[0]
    e_pad = _ceil_to(e, _STRIDE)
    src_p = _pad_entries(src, e_pad, 0)
    dst_p = _pad_entries(dst, e_pad, 0)
    d = x_table.shape[1]
    nc = -(-d // 512)
    base = (d // nc) // 16 * 16
    sizes = [base] * (nc - 1) + [d - base * (nc - 1)]
    outs = []
    c0 = 0
    for dc in sizes:
        xc = lax.slice_in_dim(x_table, c0, c0 + dc, axis=1)
        outs.append(_sc_diff_builder(x_table.shape[0], dc, e_pad)(
            xc, src_p, dst_p))
        c0 += dc
    return jnp.concatenate(outs, axis=-1) if len(outs) > 1 else outs[0]


def _msg_post(x, w, pos, neg, chunks, counts, act):
    n_pad, d = x.shape
    fn = _msg_post_builder(n_pad, d, w.shape[1], chunks, act)
    if chunks is None:
        return fn(x, pos, w)
    return fn(x, *pos, *neg, counts, w)


def _laguerre_bn(x, ei, ew, w0, w1, n_true):
    n_pad, d = x.shape
    parts, chunks = _sc_scatter(x, ei[0], ei[1], ew, n_true)
    y, st = _laguerre_builder(n_pad, n_true, d, w0.shape[1], chunks)(
        x, *parts, w0, w1)
    return _bn_relu(y, st, n_true)


def _scatter_mean(x_table, n_rows, idx, counts, n_out):
    ar = jnp.arange(n_rows, dtype=I32)
    parts, chunks = _sc_scatter(x_table, ar, idx, None, n_out)
    n_pad = parts[0].shape[1]
    outs = [_mean_combine_builder(n_pad, dc)(p, counts)
            for p, dc in zip(parts, chunks)]
    return jnp.concatenate(outs, axis=-1) if len(outs) > 1 else outs[0]


def kernel(x_t, x_s, edge_weight_t, edge_weight_s, edge_weight_t1,
           edge_weight_s1, params, edge_index_t, edge_index_s, edge_index,
           edge_index_t1, edge_index_s1, edge_index1, pos_t, pos_s,
           n_batch1, s_batch1):
    p = params
    filters = [64, 128, 256, 512]
    channels = [2, 2, 2, 2]
    n0 = x_t.shape[0]
    e0 = x_s.shape[0]
    n1 = edge_index_t1.shape[1] // 3 * 0 + 2000
    e1 = 2000
    ngraph = 64
    n0_pad = _ceil_to(n0 + 1, _ROW_PAD)
    e0_pad = _ceil_to(e0 + 1, _ROW_PAD)

    kt = _ceil_to(x_t.shape[1], 128)
    ks = _ceil_to(x_s.shape[1], 128)
    xtp = jnp.pad(x_t, ((0, n0_pad - n0), (0, kt - x_t.shape[1])))
    xsp = jnp.pad(x_s, ((0, e0_pad - e0), (0, ks - x_s.shape[1])))
    wt0 = jnp.pad(p["init_Wt"], ((0, kt - p["init_Wt"].shape[0]), (0, 0)))
    ws0 = jnp.pad(p["init_Ws"], ((0, ks - p["init_Ws"].shape[0]), (0, 0)))
    y, st = _mm_stats_builder(n0_pad, n0, kt, 64)(xtp, wt0)
    xt = _bn_relu(y, st, n0)
    y, st = _mm_stats_builder(e0_pad, e0, ks, 64)(xsp, ws0)
    xs = _bn_relu(y, st, e0)

    xt0, xs0 = xt, xs
    ei_t, ew_t = edge_index_t, edge_weight_t
    ei_s, ew_s = edge_index_s, edge_weight_s
    inc = edge_index
    nt, ne = n0, e0
    deg = _sc_bincount(inc.reshape(-1), nt)

    for i, f in enumerate(filters):
        for j in range(channels[i]):
            wt = p["int%d%d_Wt" % (i, j)]
            ws = p["int%d%d_Ws" % (i, j)]
            zs = _mm_builder(xs0.shape[0], xs0.shape[1], wt.shape[1])(xs0, wt)
            zt = _mm_builder(xt0.shape[0], xt0.shape[1], ws.shape[1])(xt0, ws)
            pos, neg, ch = _messages_z(zs, inc, nt)
            g = _gather_diff(zt, inc, ne)
            xt_i = _msg_post(xt0, wt, pos, neg, ch, deg, "relu")
            xs_i = _msg_post(xs0, ws, g, None, None, None, "relu")
            xt = _laguerre_bn(xt_i, ei_t, ew_t,
                              p["convt%d%d_W0" % (i, j)],
                              p["convt%d%d_W1" % (i, j)], nt)
            xs = _laguerre_bn(xs_i, ei_s, ew_s,
                              p["convs%d%d_W0" % (i, j)],
                              p["convs%d%d_W1" % (i, j)], ne)
            xt0 = jnp.concatenate([xt0, xt], axis=-1)
            xs0 = jnp.concatenate([xs0, xs], axis=-1)
        if i == 0:
            wat, was = p["att_Wt"], p["att_Ws"]
            zs = _mm_builder(xs0.shape[0], xs0.shape[1], wat.shape[1])(xs0, wat)
            zt = _mm_builder(xt0.shape[0], xt0.shape[1], was.shape[1])(xt0, was)
            pos, neg, ch = _messages_z(zs, inc, nt)
            g = _gather_diff(zt, inc, ne)
            at = _msg_post(xt0, wat, pos, neg, ch, deg, "attsig")
            as_ = _msg_post(xs0, was, g, None, None, None, "attsig")
            cnt_t = _sc_bincount(pos_t, n1)
            cnt_s = _sc_bincount(pos_s, e1)
            xt0 = _scatter_mean(at, nt, pos_t, cnt_t, n1)
            xs0 = _scatter_mean(as_, ne, pos_s, cnt_s, e1)
            ei_t, ew_t = edge_index_t1, edge_weight_t1
            ei_s, ew_s = edge_index_s1, edge_weight_s1
            inc = edge_index1
            nt, ne = n1, e1
            deg = _sc_bincount(inc.reshape(-1), nt)

    cnt_nb = _sc_bincount(n_batch1, ngraph)
    cnt_sb = _sc_bincount(s_batch1, ngraph)
    g_s = _scatter_mean(xs, ne, s_batch1, cnt_sb, ngraph)
    g_t = _scatter_mean(xt, nt, n_batch1, cnt_nb, ngraph)
    xg = jnp.concatenate([g_s, g_t], axis=-1)

    wout = p["out_W"]
    bout = jnp.broadcast_to(p["out_b"][None, :], (8, wout.shape[1]))
    return _head_builder(wout.shape[0], wout.shape[1])(xg, wout, bout)


# speedup vs baseline: 3.2400x; 1.0761x over previous
import functools

import jax
import jax.numpy as jnp
from jax import lax
from jax.experimental import pallas as pl
from jax.experimental.pallas import tpu as pltpu
from jax.experimental.pallas import tpu_sc as plsc

F32 = jnp.float32
I32 = jnp.int32

_CH = 64
_NW = 32
_STRIDE = _NW * _CH
_ROW_PAD = 1024
_SPMEM_BUDGET = 4 * 1024 * 1024


def _ceil_to(x, m):
    return -(-x // m) * m


def _feat_chunks(d, n_pad):
    max_dc = min(512, (7_200_000 // (4 * n_pad + 4096)) // 16 * 16)
    nc = -(-d // max_dc)
    base = (d // nc) // 16 * 16
    sizes = [base] * (nc - 1) + [d - base * (nc - 1)]
    assert all(16 <= s <= max_dc and s % 16 == 0 for s in sizes), (d, n_pad, sizes)
    return tuple(sizes)


@functools.lru_cache(maxsize=None)
def _sc_scatter_builder(nx, dc, nnz_pad, n_pad, mode, half=0):
    nchunk = nnz_pad // (_NW * _CH)
    rows_per_tile = n_pad // 16
    n_copies = rows_per_tile // _CH
    ncol = dc // 16
    mesh = plsc.VectorSubcoreMesh(core_axis_name="c", subcore_axis_name="s")

    scratch = []
    if mode != "ones":
        scratch.append(pltpu.VMEM((_CH,), I32))
    scratch.append(pltpu.VMEM((_CH,), I32))
    if mode == "gs":
        scratch.append(pltpu.VMEM((_CH, 16), F32))
    scratch += [
        pltpu.VMEM((_CH, dc), F32),
        pltpu.VMEM_SHARED((n_pad, dc), F32),
        pltpu.SemaphoreType.DMA,
    ]

    def kern(*args):
        if mode == "gs":
            (x_hbm, gidx_hbm, dst_hbm, w_hbm, out_hbm,
             gidx_v, dst_v, w_v, rows_v, acc, sem) = args
        elif mode in ("g", "pm"):
            (x_hbm, gidx_hbm, dst_hbm, out_hbm,
             gidx_v, dst_v, rows_v, acc, sem) = args
        else:
            (dst_hbm, out_hbm, dst_v, rows_v, acc, sem) = args
        c = lax.axis_index("c")
        s = lax.axis_index("s")
        wid = s * 2 + c
        t0 = s * rows_per_tile

        def fill(val):
            vec = jnp.full((16,), val, F32)

            def row(r, carry):
                for k in range(ncol):
                    rows_v[r, pl.ds(16 * k, 16)] = vec
                return carry

            lax.fori_loop(0, _CH, row, 0)

        fill(0.0)
        for j in range(n_copies):
            pltpu.sync_copy(rows_v, acc.at[pl.ds(t0 + j * _CH, _CH)])
        plsc.subcore_barrier()
        if mode == "ones":
            fill(1.0)

        def chunk(ci, carry):
            base = (wid * nchunk + ci) * _CH
            pltpu.sync_copy(dst_hbm.at[pl.ds(base, _CH)], dst_v)
            if mode != "ones":
                pltpu.sync_copy(gidx_hbm.at[pl.ds(base, _CH)], gidx_v)
                pltpu.async_copy(x_hbm.at[gidx_v], rows_v, sem).wait()
            if mode == "gs":
                pltpu.sync_copy(w_hbm.at[pl.ds(base, _CH)], w_v)

                def sgrp(g, carry2):
                    for r8 in range(8):
                        r = g * 8 + r8
                        ws = w_v[r, pl.ds(0, 16)]
                        for k in range(ncol):
                            sl = pl.ds(16 * k, 16)
                            rows_v[r, sl] = rows_v[r, sl] * ws
                    return carry2

                lax.fori_loop(0, _CH // 8, sgrp, 0)
            if mode == "pm":
                sgn = jnp.where(base >= half, F32(-1.0), F32(1.0))
                vec = jnp.full((16,), 1.0, F32) * sgn

                def ngrp(g, carry2):
                    for r8 in range(8):
                        r = g * 8 + r8
                        for k in range(ncol):
                            sl = pl.ds(16 * k, 16)
                            rows_v[r, sl] = rows_v[r, sl] * vec
                    return carry2

                lax.fori_loop(0, _CH // 8, ngrp, 0)
            pltpu.sync_copy(rows_v, acc.at[dst_v], add=True)
            return carry

        lax.fori_loop(0, nchunk, chunk, 0)
        plsc.subcore_barrier()
        for j in range(n_copies):
            sl = pl.ds(t0 + j * _CH, _CH)
            pltpu.sync_copy(acc.at[sl], out_hbm.at[c, sl])

    return pl.kernel(
        kern,
        out_type=jax.ShapeDtypeStruct((2, n_pad, dc), F32),
        mesh=mesh,
        scratch_types=scratch,
        compiler_params=pltpu.CompilerParams(use_tc_tiling_on_sc=False),
    )


@functools.lru_cache(maxsize=None)
def _sc_diff_builder(nx, dc, e_pad):
    nchunk = e_pad // (_NW * _CH)
    ncol = dc // 16
    mesh = plsc.VectorSubcoreMesh(core_axis_name="c", subcore_axis_name="s")

    def kern(x_hbm, src_hbm, dst_hbm, out_hbm, si_v, di_v, ra_v, rb_v,
             sem_a, sem_b):
        c = lax.axis_index("c")
        s = lax.axis_index("s")
        wid = s * 2 + c

        def chunk(ci, carry):
            base = (wid * nchunk + ci) * _CH
            pltpu.sync_copy(dst_hbm.at[pl.ds(base, _CH)], di_v)
            pltpu.sync_copy(src_hbm.at[pl.ds(base, _CH)], si_v)
            ca = pltpu.async_copy(x_hbm.at[di_v], ra_v, sem_a)
            cb = pltpu.async_copy(x_hbm.at[si_v], rb_v, sem_b)
            ca.wait()
            cb.wait()

            def grp(g, carry2):
                for r8 in range(8):
                    r = g * 8 + r8
                    for k in range(ncol):
                        sl = pl.ds(16 * k, 16)
                        ra_v[r, sl] = ra_v[r, sl] - rb_v[r, sl]
                return carry2

            lax.fori_loop(0, _CH // 8, grp, 0)
            pltpu.sync_copy(ra_v, out_hbm.at[pl.ds(base, _CH)])
            return carry

        lax.fori_loop(0, nchunk, chunk, 0)

    return pl.kernel(
        kern,
        out_type=jax.ShapeDtypeStruct((e_pad, dc), F32),
        mesh=mesh,
        scratch_types=[
            pltpu.VMEM((_CH,), I32), pltpu.VMEM((_CH,), I32),
            pltpu.VMEM((_CH, dc), F32), pltpu.VMEM((_CH, dc), F32),
            pltpu.SemaphoreType.DMA, pltpu.SemaphoreType.DMA,
        ],
        compiler_params=pltpu.CompilerParams(use_tc_tiling_on_sc=False),
    )


def _pad_entries(arr, nnz_pad, value):
    n = arr.shape[0]
    if n == nnz_pad:
        return arr
    return jnp.concatenate([arr, jnp.full((nnz_pad - n,), value, arr.dtype)])


def _sc_scatter(x, gidx, dst, w, n_out):
    nnz = dst.shape[0]
    nnz_pad = _ceil_to(nnz, _STRIDE)
    n_pad = _ceil_to(n_out + 1, _ROW_PAD)
    gidx_p = _pad_entries(gidx, nnz_pad, 0)
    dst_p = _pad_entries(dst, nnz_pad, n_out)
    w_p = None
    if w is not None:
        w_p = jnp.broadcast_to(_pad_entries(w, nnz_pad, 0.0)[:, None],
                               (nnz_pad, 16))
    d = x.shape[1]
    parts = []
    c0 = 0
    chunks = _feat_chunks(d, n_pad)
    for dc in chunks:
        xc = lax.slice_in_dim(x, c0, c0 + dc, axis=1)
        if w is None:
            fn = _sc_scatter_builder(x.shape[0], dc, nnz_pad, n_pad, "g")
            parts.append(fn(xc, gidx_p, dst_p))
        else:
            fn = _sc_scatter_builder(x.shape[0], dc, nnz_pad, n_pad, "gs")
            parts.append(fn(xc, gidx_p, dst_p, w_p))
        c0 += dc
    return parts, chunks


def _sc_bincount(idx, n_out):
    nnz = idx.shape[0]
    nnz_pad = _ceil_to(nnz, _STRIDE)
    n_pad = _ceil_to(n_out + 1, _ROW_PAD)
    dst_p = _pad_entries(idx, nnz_pad, n_out)
    fn = _sc_scatter_builder(0, 16, nnz_pad, n_pad, "ones")
    return fn(dst_p)


def _row_block(n_pad, d_tot):
    br = 2048 if n_pad % 2048 == 0 else n_pad
    if d_tot >= 704 and br > 1024:
        br = 1024
    return br


@functools.lru_cache(maxsize=None)
def _mm_stats_builder(n_pad, n_true, k, f):
    br = _row_block(n_pad, k)
    grid = n_pad // br

    def body(x_ref, w_ref, y_ref, st_ref):
        y = jnp.dot(x_ref[...], w_ref[...], preferred_element_type=F32)
        y_ref[...] = y
        i = pl.program_id(0)

        @pl.when(i == 0)
        def _():
            st_ref[...] = jnp.zeros_like(st_ref)

        row = i * br + lax.broadcasted_iota(I32, (br, 1), 0)
        ym = jnp.where(row < n_true, y, 0.0)
        st_ref[0:1, :] = st_ref[0:1, :] + jnp.sum(ym, axis=0, keepdims=True)
        st_ref[1:2, :] = st_ref[1:2, :] + jnp.sum(ym * ym, axis=0, keepdims=True)

    return pl.pallas_call(
        body,
        grid=(grid,),
        in_specs=[pl.BlockSpec((br, k), lambda i: (i, 0)),
                  pl.BlockSpec((k, f), lambda i: (0, 0))],
        out_specs=[pl.BlockSpec((br, f), lambda i: (i, 0)),
                   pl.BlockSpec((8, f), lambda i: (0, 0))],
        out_shape=[jax.ShapeDtypeStruct((n_pad, f), F32),
                   jax.ShapeDtypeStruct((8, f), F32)],
    )


@functools.lru_cache(maxsize=None)
def _laguerre_builder(n_pad, n_true, d, f, chunks):
    br = _row_block(n_pad, d)
    grid = n_pad // br
    nchunks = len(chunks)

    def body(*refs):
        x_ref = refs[0]
        p_refs = refs[1:1 + nchunks]
        w0_ref, w1_ref, y_ref, st_ref = refs[1 + nchunks:]
        x = x_ref[...]
        lx = jnp.concatenate([p[0] + p[1] for p in p_refs], axis=-1) \
            if nchunks > 1 else (p_refs[0][0] + p_refs[0][1])
        y = (jnp.dot(x, w0_ref[...], preferred_element_type=F32)
             + jnp.dot(x - lx, w1_ref[...], preferred_element_type=F32))
        y_ref[...] = y
        i = pl.program_id(0)

        @pl.when(i == 0)
        def _():
            st_ref[...] = jnp.zeros_like(st_ref)

        row = i * br + lax.broadcasted_iota(I32, (br, 1), 0)
        ym = jnp.where(row < n_true, y, 0.0)
        st_ref[0:1, :] = st_ref[0:1, :] + jnp.sum(ym, axis=0, keepdims=True)
        st_ref[1:2, :] = st_ref[1:2, :] + jnp.sum(ym * ym, axis=0, keepdims=True)

    in_specs = [pl.BlockSpec((br, d), lambda i: (i, 0))]
    for dc in chunks:
        in_specs.append(pl.BlockSpec((2, br, dc), lambda i: (0, i, 0)))
    in_specs += [pl.BlockSpec((d, f), lambda i: (0, 0)),
                 pl.BlockSpec((d, f), lambda i: (0, 0))]
    return pl.pallas_call(
        body,
        grid=(grid,),
        in_specs=in_specs,
        out_specs=[pl.BlockSpec((br, f), lambda i: (i, 0)),
                   pl.BlockSpec((8, f), lambda i: (0, 0))],
        out_shape=[jax.ShapeDtypeStruct((n_pad, f), F32),
                   jax.ShapeDtypeStruct((8, f), F32)],
    )


@functools.lru_cache(maxsize=None)
def _mm_builder(n_pad, k, f):
    br = _row_block(n_pad, k)
    grid = n_pad // br

    def body(x_ref, w_ref, y_ref):
        y_ref[...] = jnp.dot(x_ref[...], w_ref[...], preferred_element_type=F32)

    return pl.pallas_call(
        body,
        grid=(grid,),
        in_specs=[pl.BlockSpec((br, k), lambda i: (i, 0)),
                  pl.BlockSpec((k, f), lambda i: (0, 0))],
        out_specs=pl.BlockSpec((br, f), lambda i: (i, 0)),
        out_shape=jax.ShapeDtypeStruct((n_pad, f), F32),
    )


@functools.lru_cache(maxsize=None)
def _msg_post_builder(n_pad, d, f, chunks, act):
    br = _row_block(n_pad, max(d, f))
    grid = n_pad // br
    nchunks = 0 if chunks is None else len(chunks)

    def body(*refs):
        x_ref = refs[0]
        if chunks is None:
            m = refs[1][...]
            w_ref, o_ref = refs[2:]
        else:
            p_refs = refs[1:1 + nchunks]
            c_ref, w_ref, o_ref = refs[1 + nchunks:]
            ms = [pp[0] + pp[1] for pp in p_refs]
            m = jnp.concatenate(ms, axis=-1) if nchunks > 1 else ms[0]
            cnt = c_ref[0, :, 0:1] + c_ref[1, :, 0:1]
            m = m / (cnt + 1e-6)
        x = x_ref[...]
        y = jnp.dot(x, w_ref[...], preferred_element_type=F32) + m
        if act == "relu":
            o_ref[...] = jnp.maximum(y, 0.0)
        else:
            o_ref[...] = x * jax.nn.sigmoid(y)

    in_specs = [pl.BlockSpec((br, d), lambda i: (i, 0))]
    if chunks is None:
        in_specs.append(pl.BlockSpec((br, f), lambda i: (i, 0)))
    else:
        for dc in chunks:
            in_specs.append(pl.BlockSpec((2, br, dc), lambda i: (0, i, 0)))
        in_specs.append(pl.BlockSpec((2, br, 16), lambda i: (0, i, 0)))
    in_specs.append(pl.BlockSpec((d, f), lambda i: (0, 0)))
    return pl.pallas_call(
        body,
        grid=(grid,),
        in_specs=in_specs,
        out_specs=pl.BlockSpec((br, f), lambda i: (i, 0)),
        out_shape=jax.ShapeDtypeStruct((n_pad, f), F32),
    )


@functools.lru_cache(maxsize=None)
def _bn_relu_builder(n_pad, n_true, f):
    br = _row_block(n_pad, f)
    grid = n_pad // br
    inv_n = 1.0 / n_true

    def body(y_ref, st_ref, o_ref):
        mu = st_ref[0:1, :] * inv_n
        var = st_ref[1:2, :] * inv_n - mu * mu
        y = jnp.maximum((y_ref[...] - mu) * lax.rsqrt(var + 1e-5), 0.0)
        row = pl.program_id(0) * br + lax.broadcasted_iota(I32, (br, 1), 0)
        o_ref[...] = jnp.where(row < n_true, y, 0.0)

    return pl.pallas_call(
        body,
        grid=(grid,),
        in_specs=[pl.BlockSpec((br, f), lambda i: (i, 0)),
                  pl.BlockSpec((8, f), lambda i: (0, 0))],
        out_specs=pl.BlockSpec((br, f), lambda i: (i, 0)),
        out_shape=jax.ShapeDtypeStruct((n_pad, f), F32),
    )


@functools.lru_cache(maxsize=None)
def _mean_combine_builder(n_pad, dc):
    br = 2048 if n_pad % 2048 == 0 else n_pad
    grid = n_pad // br

    def body(p_ref, c_ref, o_ref):
        cnt = c_ref[0, :, 0:1] + c_ref[1, :, 0:1]
        o_ref[...] = (p_ref[0] + p_ref[1]) / jnp.maximum(cnt, 1.0)

    return pl.pallas_call(
        body,
        grid=(grid,),
        in_specs=[pl.BlockSpec((2, br, dc), lambda i: (0, i, 0)),
                  pl.BlockSpec((2, br, 16), lambda i: (0, i, 0))],
        out_specs=pl.BlockSpec((br, dc), lambda i: (i, 0)),
        out_shape=jax.ShapeDtypeStruct((n_pad, dc), F32),
    )


@functools.lru_cache(maxsize=None)
def _head_builder(k, f):
    def body(x_ref, w_ref, b_ref, o_ref):
        o_ref[...] = (jnp.dot(x_ref[...], w_ref[...], preferred_element_type=F32)
                      + b_ref[0:1, :])

    return pl.pallas_call(
        body,
        grid=(1,),
        in_specs=[pl.BlockSpec((64, k), lambda i: (0, 0)),
                  pl.BlockSpec((k, f), lambda i: (0, 0)),
                  pl.BlockSpec((8, f), lambda i: (0, 0))],
        out_specs=pl.BlockSpec((64, f), lambda i: (0, 0)),
        out_shape=jax.ShapeDtypeStruct((64, f), F32),
    )


def _bn_relu(y, st, n_true):
    return _bn_relu_builder(y.shape[0], n_true, y.shape[1])(y, st)


def _messages_z(z, inc, n_out):
    e = inc.shape[1]
    half = _ceil_to(e, _STRIDE // 2)
    ar = jnp.arange(e, dtype=I32)
    ar_p = _pad_entries(ar, half, 0)
    gidx = jnp.concatenate([ar_p, ar_p])
    dsts = jnp.concatenate([_pad_entries(inc[1], half, n_out),
                            _pad_entries(inc[0], half, n_out)])
    n_pad = _ceil_to(n_out + 1, _ROW_PAD)
    parts = []
    c0 = 0
    chunks = _feat_chunks(z.shape[1], n_pad)
    for dc in chunks:
        zc = lax.slice_in_dim(z, c0, c0 + dc, axis=1)
        fn = _sc_scatter_builder(z.shape[0], dc, 2 * half, n_pad, "pm", half)
        parts.append(fn(zc, gidx, dsts))
        c0 += dc
    return parts, chunks


def _gather_diff(x_table, inc, n_out):
    src, dst = inc[0], inc[1]
    e = src.shape[0]
    e_pad = _ceil_to(e, _STRIDE)
    src_p = _pad_entries(src, e_pad, 0)
    dst_p = _pad_entries(dst, e_pad, 0)
    d = x_table.shape[1]
    nc = -(-d // 512)
    base = (d // nc) // 16 * 16
    sizes = [base] * (nc - 1) + [d - base * (nc - 1)]
    outs = []
    c0 = 0
    for dc in sizes:
        xc = lax.slice_in_dim(x_table, c0, c0 + dc, axis=1)
        outs.append(_sc_diff_builder(x_table.shape[0], dc, e_pad)(
            xc, src_p, dst_p))
        c0 += dc
    return jnp.concatenate(outs, axis=-1) if len(outs) > 1 else outs[0]


def _msg_post(x, w, parts, chunks, counts, act):
    n_pad, d = x.shape
    fn = _msg_post_builder(n_pad, d, w.shape[1], chunks, act)
    if chunks is None:
        return fn(x, parts, w)
    return fn(x, *parts, counts, w)


def _laguerre_bn(x, ei, ew, w0, w1, n_true):
    n_pad, d = x.shape
    parts, chunks = _sc_scatter(x, ei[0], ei[1], ew, n_true)
    y, st = _laguerre_builder(n_pad, n_true, d, w0.shape[1], chunks)(
        x, *parts, w0, w1)
    return _bn_relu(y, st, n_true)


def _scatter_mean(x_table, n_rows, idx, counts, n_out):
    ar = jnp.arange(n_rows, dtype=I32)
    parts, chunks = _sc_scatter(x_table, ar, idx, None, n_out)
    n_pad = parts[0].shape[1]
    outs = [_mean_combine_builder(n_pad, dc)(p, counts)
            for p, dc in zip(parts, chunks)]
    return jnp.concatenate(outs, axis=-1) if len(outs) > 1 else outs[0]


def kernel(x_t, x_s, edge_weight_t, edge_weight_s, edge_weight_t1,
           edge_weight_s1, params, edge_index_t, edge_index_s, edge_index,
           edge_index_t1, edge_index_s1, edge_index1, pos_t, pos_s,
           n_batch1, s_batch1):
    p = params
    filters = [64, 128, 256, 512]
    channels = [2, 2, 2, 2]
    n0 = x_t.shape[0]
    e0 = x_s.shape[0]
    n1 = edge_index_t1.shape[1] // 3 * 0 + 2000
    e1 = 2000
    ngraph = 64
    n0_pad = _ceil_to(n0 + 1, _ROW_PAD)
    e0_pad = _ceil_to(e0 + 1, _ROW_PAD)

    kt = _ceil_to(x_t.shape[1], 128)
    ks = _ceil_to(x_s.shape[1], 128)
    xtp = jnp.pad(x_t, ((0, n0_pad - n0), (0, kt - x_t.shape[1])))
    xsp = jnp.pad(x_s, ((0, e0_pad - e0), (0, ks - x_s.shape[1])))
    wt0 = jnp.pad(p["init_Wt"], ((0, kt - p["init_Wt"].shape[0]), (0, 0)))
    ws0 = jnp.pad(p["init_Ws"], ((0, ks - p["init_Ws"].shape[0]), (0, 0)))
    y, st = _mm_stats_builder(n0_pad, n0, kt, 64)(xtp, wt0)
    xt = _bn_relu(y, st, n0)
    y, st = _mm_stats_builder(e0_pad, e0, ks, 64)(xsp, ws0)
    xs = _bn_relu(y, st, e0)

    xt0, xs0 = xt, xs
    ei_t, ew_t = edge_index_t, edge_weight_t
    ei_s, ew_s = edge_index_s, edge_weight_s
    inc = edge_index
    nt, ne = n0, e0
    deg = _sc_bincount(inc.reshape(-1), nt)

    for i, f in enumerate(filters):
        for j in range(channels[i]):
            wt = p["int%d%d_Wt" % (i, j)]
            ws = p["int%d%d_Ws" % (i, j)]
            zs = _mm_builder(xs0.shape[0], xs0.shape[1], wt.shape[1])(xs0, wt)
            zt = _mm_builder(xt0.shape[0], xt0.shape[1], ws.shape[1])(xt0, ws)
            mt_parts, ch = _messages_z(zs, inc, nt)
            g = _gather_diff(zt, inc, ne)
            xt_i = _msg_post(xt0, wt, mt_parts, ch, deg, "relu")
            xs_i = _msg_post(xs0, ws, g, None, None, "relu")
            xt = _laguerre_bn(xt_i, ei_t, ew_t,
                              p["convt%d%d_W0" % (i, j)],
                              p["convt%d%d_W1" % (i, j)], nt)
            xs = _laguerre_bn(xs_i, ei_s, ew_s,
                              p["convs%d%d_W0" % (i, j)],
                              p["convs%d%d_W1" % (i, j)], ne)
            xt0 = jnp.concatenate([xt0, xt], axis=-1)
            xs0 = jnp.concatenate([xs0, xs], axis=-1)
        if i == 0:
            wat, was = p["att_Wt"], p["att_Ws"]
            zs = _mm_builder(xs0.shape[0], xs0.shape[1], wat.shape[1])(xs0, wat)
            zt = _mm_builder(xt0.shape[0], xt0.shape[1], was.shape[1])(xt0, was)
            mt_parts, ch = _messages_z(zs, inc, nt)
            g = _gather_diff(zt, inc, ne)
            at = _msg_post(xt0, wat, mt_parts, ch, deg, "attsig")
            as_ = _msg_post(xs0, was, g, None, None, "attsig")
            cnt_t = _sc_bincount(pos_t, n1)
            cnt_s = _sc_bincount(pos_s, e1)
            xt0 = _scatter_mean(at, nt, pos_t, cnt_t, n1)
            xs0 = _scatter_mean(as_, ne, pos_s, cnt_s, e1)
            ei_t, ew_t = edge_index_t1, edge_weight_t1
            ei_s, ew_s = edge_index_s1, edge_weight_s1
            inc = edge_index1
            nt, ne = n1, e1
            deg = _sc_bincount(inc.reshape(-1), nt)

    cnt_nb = _sc_bincount(n_batch1, ngraph)
    cnt_sb = _sc_bincount(s_batch1, ngraph)
    g_s = _scatter_mean(xs, ne, s_batch1, cnt_sb, ngraph)
    g_t = _scatter_mean(xt, nt, n_batch1, cnt_nb, ngraph)
    xg = jnp.concatenate([g_s, g_t], axis=-1)

    wout = p["out_W"]
    bout = jnp.broadcast_to(p["out_b"][None, :], (8, wout.shape[1]))
    return _head_builder(wout.shape[0], wout.shape[1])(xg, wout, bout)


# speedup vs baseline: 3.4140x; 1.0537x over previous
import functools

import jax
import jax.numpy as jnp
from jax import lax
from jax.experimental import pallas as pl
from jax.experimental.pallas import tpu as pltpu
from jax.experimental.pallas import tpu_sc as plsc

F32 = jnp.float32
I32 = jnp.int32

_CH = 64
_NW = 32
_STRIDE = _NW * _CH
_ROW_PAD = 1024
_SPMEM_BUDGET = 4 * 1024 * 1024


def _ceil_to(x, m):
    return -(-x // m) * m


def _feat_chunks(d, n_pad):
    max_dc = min(512, (7_200_000 // (4 * n_pad + 8192)) // 16 * 16)
    nc = -(-d // max_dc)
    base = (d // nc) // 16 * 16
    sizes = [base] * (nc - 1) + [d - base * (nc - 1)]
    assert all(16 <= s <= max_dc and s % 16 == 0 for s in sizes), (d, n_pad, sizes)
    return tuple(sizes)


@functools.lru_cache(maxsize=None)
def _sc_scatter_builder(nx, dc, nnz_pad, n_pad, mode, half=0):
    nchunk = nnz_pad // (_NW * _CH)
    rows_per_tile = n_pad // 16
    n_copies = rows_per_tile // _CH
    ncol = dc // 16
    mesh = plsc.VectorSubcoreMesh(core_axis_name="c", subcore_axis_name="s")

    scratch = []
    if mode == "ones":
        scratch += [
            pltpu.VMEM((_CH,), I32),
            pltpu.VMEM((_CH, dc), F32),
            pltpu.VMEM_SHARED((n_pad, dc), F32),
        ]
    else:
        scratch += [pltpu.VMEM((_CH,), I32)] * 2
        scratch += [pltpu.VMEM((_CH,), I32)] * 2
        if mode == "gs":
            scratch += [pltpu.VMEM((_CH, 16), F32)] * 2
        scratch += [pltpu.VMEM((_CH, dc), F32)] * 2
        scratch.append(pltpu.VMEM_SHARED((n_pad, dc), F32))
        nsem = 8 if mode == "gs" else 6
        scratch += [pltpu.SemaphoreType.DMA] * nsem

    def kern(*args):
        if mode == "ones":
            (dst_hbm, out_hbm, dst_v, rows_v, acc) = args
        elif mode == "gs":
            (x_hbm, gidx_hbm, dst_hbm, w_hbm, out_hbm,
             g0, g1, d0, d1, w0, w1, r0, r1, acc,
             sg0, sg1, sd0, sd1, sw0, sw1, sr0, sr1) = args
            gv, dv, wv, rv = (g0, g1), (d0, d1), (w0, w1), (r0, r1)
            sg, sd, sw, sr = (sg0, sg1), (sd0, sd1), (sw0, sw1), (sr0, sr1)
        else:
            (x_hbm, gidx_hbm, dst_hbm, out_hbm,
             g0, g1, d0, d1, r0, r1, acc,
             sg0, sg1, sd0, sd1, sr0, sr1) = args
            gv, dv, rv = (g0, g1), (d0, d1), (r0, r1)
            sg, sd, sr = (sg0, sg1), (sd0, sd1), (sr0, sr1)
        c = lax.axis_index("c")
        s = lax.axis_index("s")
        wid = s * 2 + c
        t0 = s * rows_per_tile

        def fill(buf, val):
            vec = jnp.full((16,), val, F32)

            def row(r, carry):
                for k in range(ncol):
                    buf[r, pl.ds(16 * k, 16)] = vec
                return carry

            lax.fori_loop(0, _CH, row, 0)

        zbuf = rows_v if mode == "ones" else r0
        fill(zbuf, 0.0)
        for j in range(n_copies):
            pltpu.sync_copy(zbuf, acc.at[pl.ds(t0 + j * _CH, _CH)])
        plsc.subcore_barrier()

        if mode == "ones":
            fill(rows_v, 1.0)

            def chunk(ci, carry):
                base = (wid * nchunk + ci) * _CH
                pltpu.sync_copy(dst_hbm.at[pl.ds(base, _CH)], dst_v)
                pltpu.sync_copy(rows_v, acc.at[dst_v], add=True)
                return carry

            lax.fori_loop(0, nchunk, chunk, 0)
        else:
            handles = [None, None]

            def start_chunk(ci):
                b = ci % 2
                base = (wid * nchunk + ci) * _CH
                h = {
                    "g": pltpu.async_copy(
                        gidx_hbm.at[pl.ds(base, _CH)], gv[b], sg[b]),
                    "d": pltpu.async_copy(
                        dst_hbm.at[pl.ds(base, _CH)], dv[b], sd[b]),
                }
                if mode == "gs":
                    h["w"] = pltpu.async_copy(
                        w_hbm.at[pl.ds(base, _CH)], wv[b], sw[b])
                handles[b] = h

            def start_gather(ci):
                b = ci % 2
                handles[b]["g"].wait()
                handles[b]["r"] = pltpu.async_copy(x_hbm.at[gv[b]], rv[b],
                                                   sr[b])

            def process(ci):
                b = ci % 2
                base = (wid * nchunk + ci) * _CH
                handles[b]["r"].wait()
                if mode == "gs":
                    handles[b]["w"].wait()

                    def sgrp(g, carry2):
                        for r8 in range(8):
                            r = g * 8 + r8
                            ws = wv[b][r, pl.ds(0, 16)]
                            for k in range(ncol):
                                sl = pl.ds(16 * k, 16)
                                rv[b][r, sl] = rv[b][r, sl] * ws
                        return carry2

                    lax.fori_loop(0, _CH // 8, sgrp, 0)
                if mode == "pm":
                    sgn = jnp.where(base >= half, F32(-1.0), F32(1.0))
                    vec = jnp.full((16,), 1.0, F32) * sgn

                    def ngrp(g, carry2):
                        for r8 in range(8):
                            r = g * 8 + r8
                            for k in range(ncol):
                                sl = pl.ds(16 * k, 16)
                                rv[b][r, sl] = rv[b][r, sl] * vec
                        return carry2

                    lax.fori_loop(0, _CH // 8, ngrp, 0)
                handles[b]["d"].wait()
                pltpu.sync_copy(rv[b], acc.at[dv[b]], add=True)

            start_chunk(0)
            start_gather(0)
            for ci in range(nchunk):
                if ci + 1 < nchunk:
                    start_chunk(ci + 1)
                    start_gather(ci + 1)
                process(ci)
        plsc.subcore_barrier()
        for j in range(n_copies):
            sl = pl.ds(t0 + j * _CH, _CH)
            pltpu.sync_copy(acc.at[sl], out_hbm.at[c, sl])

    return pl.kernel(
        kern,
        out_type=jax.ShapeDtypeStruct((2, n_pad, dc), F32),
        mesh=mesh,
        scratch_types=scratch,
        compiler_params=pltpu.CompilerParams(use_tc_tiling_on_sc=False),
    )


@functools.lru_cache(maxsize=None)
def _sc_diff_builder(nx, dc, e_pad):
    nchunk = e_pad // (_NW * _CH)
    ncol = dc // 16
    mesh = plsc.VectorSubcoreMesh(core_axis_name="c", subcore_axis_name="s")

    def kern(x_hbm, src_hbm, dst_hbm, out_hbm,
             s0, s1, d0, d1, ra0, ra1, rb0, rb1,
             ss0, ss1, sd0, sd1, sa0, sa1, sb0, sb1):
        c = lax.axis_index("c")
        s = lax.axis_index("s")
        wid = s * 2 + c
        sv, dv = (s0, s1), (d0, d1)
        rav, rbv = (ra0, ra1), (rb0, rb1)
        ssem, dsem = (ss0, ss1), (sd0, sd1)
        asem, bsem = (sa0, sa1), (sb0, sb1)
        handles = [None, None]

        def start_chunk(ci):
            b = ci % 2
            base = (wid * nchunk + ci) * _CH
            handles[b] = {
                "s": pltpu.async_copy(src_hbm.at[pl.ds(base, _CH)], sv[b],
                                      ssem[b]),
                "d": pltpu.async_copy(dst_hbm.at[pl.ds(base, _CH)], dv[b],
                                      dsem[b]),
            }

        def start_gather(ci):
            b = ci % 2
            h = handles[b]
            h["d"].wait()
            h["a"] = pltpu.async_copy(x_hbm.at[dv[b]], rav[b], asem[b])
            h["s"].wait()
            h["b"] = pltpu.async_copy(x_hbm.at[sv[b]], rbv[b], bsem[b])

        def process(ci):
            b = ci % 2
            base = (wid * nchunk + ci) * _CH
            handles[b]["a"].wait()
            handles[b]["b"].wait()

            def grp(g, carry2):
                for r8 in range(8):
                    r = g * 8 + r8
                    for k in range(ncol):
                        sl = pl.ds(16 * k, 16)
                        rav[b][r, sl] = rav[b][r, sl] - rbv[b][r, sl]
                return carry2

            lax.fori_loop(0, _CH // 8, grp, 0)
            pltpu.sync_copy(rav[b], out_hbm.at[pl.ds(base, _CH)])

        start_chunk(0)
        start_gather(0)
        for ci in range(nchunk):
            if ci + 1 < nchunk:
                start_chunk(ci + 1)
                start_gather(ci + 1)
            process(ci)

    return pl.kernel(
        kern,
        out_type=jax.ShapeDtypeStruct((e_pad, dc), F32),
        mesh=mesh,
        scratch_types=(
            [pltpu.VMEM((_CH,), I32)] * 4
            + [pltpu.VMEM((_CH, dc), F32)] * 4
            + [pltpu.SemaphoreType.DMA] * 8
        ),
        compiler_params=pltpu.CompilerParams(use_tc_tiling_on_sc=False),
    )


def _pad_entries(arr, nnz_pad, value):
    n = arr.shape[0]
    if n == nnz_pad:
        return arr
    return jnp.concatenate([arr, jnp.full((nnz_pad - n,), value, arr.dtype)])


def _sc_scatter(x, gidx, dst, w, n_out):
    nnz = dst.shape[0]
    nnz_pad = _ceil_to(nnz, _STRIDE)
    n_pad = _ceil_to(n_out + 1, _ROW_PAD)
    gidx_p = _pad_entries(gidx, nnz_pad, 0)
    dst_p = _pad_entries(dst, nnz_pad, n_out)
    w_p = None
    if w is not None:
        w_p = jnp.broadcast_to(_pad_entries(w, nnz_pad, 0.0)[:, None],
                               (nnz_pad, 16))
    d = x.shape[1]
    parts = []
    c0 = 0
    chunks = _feat_chunks(d, n_pad)
    for dc in chunks:
        xc = lax.slice_in_dim(x, c0, c0 + dc, axis=1)
        if w is None:
            fn = _sc_scatter_builder(x.shape[0], dc, nnz_pad, n_pad, "g")
            parts.append(fn(xc, gidx_p, dst_p))
        else:
            fn = _sc_scatter_builder(x.shape[0], dc, nnz_pad, n_pad, "gs")
            parts.append(fn(xc, gidx_p, dst_p, w_p))
        c0 += dc
    return parts, chunks


def _sc_bincount(idx, n_out):
    nnz = idx.shape[0]
    nnz_pad = _ceil_to(nnz, _STRIDE)
    n_pad = _ceil_to(n_out + 1, _ROW_PAD)
    dst_p = _pad_entries(idx, nnz_pad, n_out)
    fn = _sc_scatter_builder(0, 16, nnz_pad, n_pad, "ones")
    return fn(dst_p)


def _row_block(n_pad, d_tot):
    br = 2048 if n_pad % 2048 == 0 else n_pad
    if d_tot >= 704 and br > 1024:
        br = 1024
    return br


@functools.lru_cache(maxsize=None)
def _mm_stats_builder(n_pad, n_true, k, f):
    br = _row_block(n_pad, k)
    grid = n_pad // br

    def body(x_ref, w_ref, y_ref, st_ref):
        y = jnp.dot(x_ref[...], w_ref[...], preferred_element_type=F32)
        y_ref[...] = y
        i = pl.program_id(0)

        @pl.when(i == 0)
        def _():
            st_ref[...] = jnp.zeros_like(st_ref)

        row = i * br + lax.broadcasted_iota(I32, (br, 1), 0)
        ym = jnp.where(row < n_true, y, 0.0)
        st_ref[0:1, :] = st_ref[0:1, :] + jnp.sum(ym, axis=0, keepdims=True)
        st_ref[1:2, :] = st_ref[1:2, :] + jnp.sum(ym * ym, axis=0, keepdims=True)

    return pl.pallas_call(
        body,
        grid=(grid,),
        in_specs=[pl.BlockSpec((br, k), lambda i: (i, 0)),
                  pl.BlockSpec((k, f), lambda i: (0, 0))],
        out_specs=[pl.BlockSpec((br, f), lambda i: (i, 0)),
                   pl.BlockSpec((8, f), lambda i: (0, 0))],
        out_shape=[jax.ShapeDtypeStruct((n_pad, f), F32),
                   jax.ShapeDtypeStruct((8, f), F32)],
    )


@functools.lru_cache(maxsize=None)
def _laguerre_builder(n_pad, n_true, d, f, chunks):
    br = _row_block(n_pad, d)
    grid = n_pad // br
    nchunks = len(chunks)

    def body(*refs):
        x_ref = refs[0]
        p_refs = refs[1:1 + nchunks]
        w0_ref, w1_ref, y_ref, st_ref = refs[1 + nchunks:]
        x = x_ref[...]
        lx = jnp.concatenate([p[0] + p[1] for p in p_refs], axis=-1) \
            if nchunks > 1 else (p_refs[0][0] + p_refs[0][1])
        y = (jnp.dot(x, w0_ref[...], preferred_element_type=F32)
             + jnp.dot(x - lx, w1_ref[...], preferred_element_type=F32))
        y_ref[...] = y
        i = pl.program_id(0)

        @pl.when(i == 0)
        def _():
            st_ref[...] = jnp.zeros_like(st_ref)

        row = i * br + lax.broadcasted_iota(I32, (br, 1), 0)
        ym = jnp.where(row < n_true, y, 0.0)
        st_ref[0:1, :] = st_ref[0:1, :] + jnp.sum(ym, axis=0, keepdims=True)
        st_ref[1:2, :] = st_ref[1:2, :] + jnp.sum(ym * ym, axis=0, keepdims=True)

    in_specs = [pl.BlockSpec((br, d), lambda i: (i, 0))]
    for dc in chunks:
        in_specs.append(pl.BlockSpec((2, br, dc), lambda i: (0, i, 0)))
    in_specs += [pl.BlockSpec((d, f), lambda i: (0, 0)),
                 pl.BlockSpec((d, f), lambda i: (0, 0))]
    return pl.pallas_call(
        body,
        grid=(grid,),
        in_specs=in_specs,
        out_specs=[pl.BlockSpec((br, f), lambda i: (i, 0)),
                   pl.BlockSpec((8, f), lambda i: (0, 0))],
        out_shape=[jax.ShapeDtypeStruct((n_pad, f), F32),
                   jax.ShapeDtypeStruct((8, f), F32)],
    )


@functools.lru_cache(maxsize=None)
def _mm_builder(n_pad, k, f):
    br = _row_block(n_pad, k)
    grid = n_pad // br

    def body(x_ref, w_ref, y_ref):
        y_ref[...] = jnp.dot(x_ref[...], w_ref[...], preferred_element_type=F32)

    return pl.pallas_call(
        body,
        grid=(grid,),
        in_specs=[pl.BlockSpec((br, k), lambda i: (i, 0)),
                  pl.BlockSpec((k, f), lambda i: (0, 0))],
        out_specs=pl.BlockSpec((br, f), lambda i: (i, 0)),
        out_shape=jax.ShapeDtypeStruct((n_pad, f), F32),
    )


@functools.lru_cache(maxsize=None)
def _msg_post_builder(n_pad, d, f, chunks, act):
    br = _row_block(n_pad, max(d, f))
    grid = n_pad // br
    nchunks = 0 if chunks is None else len(chunks)

    def body(*refs):
        x_ref = refs[0]
        if chunks is None:
            m = refs[1][...]
            w_ref, o_ref = refs[2:]
        else:
            p_refs = refs[1:1 + nchunks]
            c_ref, w_ref, o_ref = refs[1 + nchunks:]
            ms = [pp[0] + pp[1] for pp in p_refs]
            m = jnp.concatenate(ms, axis=-1) if nchunks > 1 else ms[0]
            cnt = c_ref[0, :, 0:1] + c_ref[1, :, 0:1]
            m = m / (cnt + 1e-6)
        x = x_ref[...]
        y = jnp.dot(x, w_ref[...], preferred_element_type=F32) + m
        if act == "relu":
            o_ref[...] = jnp.maximum(y, 0.0)
        else:
            o_ref[...] = x * jax.nn.sigmoid(y)

    in_specs = [pl.BlockSpec((br, d), lambda i: (i, 0))]
    if chunks is None:
        in_specs.append(pl.BlockSpec((br, f), lambda i: (i, 0)))
    else:
        for dc in chunks:
            in_specs.append(pl.BlockSpec((2, br, dc), lambda i: (0, i, 0)))
        in_specs.append(pl.BlockSpec((2, br, 16), lambda i: (0, i, 0)))
    in_specs.append(pl.BlockSpec((d, f), lambda i: (0, 0)))
    return pl.pallas_call(
        body,
        grid=(grid,),
        in_specs=in_specs,
        out_specs=pl.BlockSpec((br, f), lambda i: (i, 0)),
        out_shape=jax.ShapeDtypeStruct((n_pad, f), F32),
    )


@functools.lru_cache(maxsize=None)
def _bn_relu_builder(n_pad, n_true, f):
    br = _row_block(n_pad, f)
    grid = n_pad // br
    inv_n = 1.0 / n_true

    def body(y_ref, st_ref, o_ref):
        mu = st_ref[0:1, :] * inv_n
        var = st_ref[1:2, :] * inv_n - mu * mu
        y = jnp.maximum((y_ref[...] - mu) * lax.rsqrt(var + 1e-5), 0.0)
        row = pl.program_id(0) * br + lax.broadcasted_iota(I32, (br, 1), 0)
        o_ref[...] = jnp.where(row < n_true, y, 0.0)

    return pl.pallas_call(
        body,
        grid=(grid,),
        in_specs=[pl.BlockSpec((br, f), lambda i: (i, 0)),
                  pl.BlockSpec((8, f), lambda i: (0, 0))],
        out_specs=pl.BlockSpec((br, f), lambda i: (i, 0)),
        out_shape=jax.ShapeDtypeStruct((n_pad, f), F32),
    )


@functools.lru_cache(maxsize=None)
def _mean_combine_builder(n_pad, dc):
    br = 2048 if n_pad % 2048 == 0 else n_pad
    grid = n_pad // br

    def body(p_ref, c_ref, o_ref):
        cnt = c_ref[0, :, 0:1] + c_ref[1, :, 0:1]
        o_ref[...] = (p_ref[0] + p_ref[1]) / jnp.maximum(cnt, 1.0)

    return pl.pallas_call(
        body,
        grid=(grid,),
        in_specs=[pl.BlockSpec((2, br, dc), lambda i: (0, i, 0)),
                  pl.BlockSpec((2, br, 16), lambda i: (0, i, 0))],
        out_specs=pl.BlockSpec((br, dc), lambda i: (i, 0)),
        out_shape=jax.ShapeDtypeStruct((n_pad, dc), F32),
    )


@functools.lru_cache(maxsize=None)
def _head_builder(k, f):
    def body(x_ref, w_ref, b_ref, o_ref):
        o_ref[...] = (jnp.dot(x_ref[...], w_ref[...], preferred_element_type=F32)
                      + b_ref[0:1, :])

    return pl.pallas_call(
        body,
        grid=(1,),
        in_specs=[pl.BlockSpec((64, k), lambda i: (0, 0)),
                  pl.BlockSpec((k, f), lambda i: (0, 0)),
                  pl.BlockSpec((8, f), lambda i: (0, 0))],
        out_specs=pl.BlockSpec((64, f), lambda i: (0, 0)),
        out_shape=jax.ShapeDtypeStruct((64, f), F32),
    )


def _bn_relu(y, st, n_true):
    return _bn_relu_builder(y.shape[0], n_true, y.shape[1])(y, st)


def _messages_z(z, inc, n_out):
    e = inc.shape[1]
    half = _ceil_to(e, _STRIDE // 2)
    ar = jnp.arange(e, dtype=I32)
    ar_p = _pad_entries(ar, half, 0)
    gidx = jnp.concatenate([ar_p, ar_p])
    dsts = jnp.concatenate([_pad_entries(inc[1], half, n_out),
                            _pad_entries(inc[0], half, n_out)])
    n_pad = _ceil_to(n_out + 1, _ROW_PAD)
    parts = []
    c0 = 0
    chunks = _feat_chunks(z.shape[1], n_pad)
    for dc in chunks:
        zc = lax.slice_in_dim(z, c0, c0 + dc, axis=1)
        fn = _sc_scatter_builder(z.shape[0], dc, 2 * half, n_pad, "pm", half)
        parts.append(fn(zc, gidx, dsts))
        c0 += dc
    return parts, chunks


def _gather_diff(x_table, inc, n_out):
    src, dst = inc[0], inc[1]
    e = src.shape[0]
    e_pad = _ceil_to(e, _STRIDE)
    src_p = _pad_entries(src, e_pad, 0)
    dst_p = _pad_entries(dst, e_pad, 0)
    d = x_table.shape[1]
    nc = -(-d // 384)
    base = (d // nc) // 16 * 16
    sizes = [base] * (nc - 1) + [d - base * (nc - 1)]
    outs = []
    c0 = 0
    for dc in sizes:
        xc = lax.slice_in_dim(x_table, c0, c0 + dc, axis=1)
        outs.append(_sc_diff_builder(x_table.shape[0], dc, e_pad)(
            xc, src_p, dst_p))
        c0 += dc
    return jnp.concatenate(outs, axis=-1) if len(outs) > 1 else outs[0]


def _msg_post(x, w, parts, chunks, counts, act):
    n_pad, d = x.shape
    fn = _msg_post_builder(n_pad, d, w.shape[1], chunks, act)
    if chunks is None:
        return fn(x, parts, w)
    return fn(x, *parts, counts, w)


def _laguerre_bn(x, ei, ew, w0, w1, n_true):
    n_pad, d = x.shape
    parts, chunks = _sc_scatter(x, ei[0], ei[1], ew, n_true)
    y, st = _laguerre_builder(n_pad, n_true, d, w0.shape[1], chunks)(
        x, *parts, w0, w1)
    return _bn_relu(y, st, n_true)


def _scatter_mean(x_table, n_rows, idx, counts, n_out):
    ar = jnp.arange(n_rows, dtype=I32)
    parts, chunks = _sc_scatter(x_table, ar, idx, None, n_out)
    n_pad = parts[0].shape[1]
    outs = [_mean_combine_builder(n_pad, dc)(p, counts)
            for p, dc in zip(parts, chunks)]
    return jnp.concatenate(outs, axis=-1) if len(outs) > 1 else outs[0]


def kernel(x_t, x_s, edge_weight_t, edge_weight_s, edge_weight_t1,
           edge_weight_s1, params, edge_index_t, edge_index_s, edge_index,
           edge_index_t1, edge_index_s1, edge_index1, pos_t, pos_s,
           n_batch1, s_batch1):
    p = params
    filters = [64, 128, 256, 512]
    channels = [2, 2, 2, 2]
    n0 = x_t.shape[0]
    e0 = x_s.shape[0]
    n1 = edge_index_t1.shape[1] // 3 * 0 + 2000
    e1 = 2000
    ngraph = 64
    n0_pad = _ceil_to(n0 + 1, _ROW_PAD)
    e0_pad = _ceil_to(e0 + 1, _ROW_PAD)

    kt = _ceil_to(x_t.shape[1], 128)
    ks = _ceil_to(x_s.shape[1], 128)
    xtp = jnp.pad(x_t, ((0, n0_pad - n0), (0, kt - x_t.shape[1])))
    xsp = jnp.pad(x_s, ((0, e0_pad - e0), (0, ks - x_s.shape[1])))
    wt0 = jnp.pad(p["init_Wt"], ((0, kt - p["init_Wt"].shape[0]), (0, 0)))
    ws0 = jnp.pad(p["init_Ws"], ((0, ks - p["init_Ws"].shape[0]), (0, 0)))
    y, st = _mm_stats_builder(n0_pad, n0, kt, 64)(xtp, wt0)
    xt = _bn_relu(y, st, n0)
    y, st = _mm_stats_builder(e0_pad, e0, ks, 64)(xsp, ws0)
    xs = _bn_relu(y, st, e0)

    xt0, xs0 = xt, xs
    ei_t, ew_t = edge_index_t, edge_weight_t
    ei_s, ew_s = edge_index_s, edge_weight_s
    inc = edge_index
    nt, ne = n0, e0
    deg = _sc_bincount(inc.reshape(-1), nt)

    for i, f in enumerate(filters):
        for j in range(channels[i]):
            wt = p["int%d%d_Wt" % (i, j)]
            ws = p["int%d%d_Ws" % (i, j)]
            zs = _mm_builder(xs0.shape[0], xs0.shape[1], wt.shape[1])(xs0, wt)
            zt = _mm_builder(xt0.shape[0], xt0.shape[1], ws.shape[1])(xt0, ws)
            mt_parts, ch = _messages_z(zs, inc, nt)
            g = _gather_diff(zt, inc, ne)
            xt_i = _msg_post(xt0, wt, mt_parts, ch, deg, "relu")
            xs_i = _msg_post(xs0, ws, g, None, None, "relu")
            xt = _laguerre_bn(xt_i, ei_t, ew_t,
                              p["convt%d%d_W0" % (i, j)],
                              p["convt%d%d_W1" % (i, j)], nt)
            xs = _laguerre_bn(xs_i, ei_s, ew_s,
                              p["convs%d%d_W0" % (i, j)],
                              p["convs%d%d_W1" % (i, j)], ne)
            xt0 = jnp.concatenate([xt0, xt], axis=-1)
            xs0 = jnp.concatenate([xs0, xs], axis=-1)
        if i == 0:
            wat, was = p["att_Wt"], p["att_Ws"]
            zs = _mm_builder(xs0.shape[0], xs0.shape[1], wat.shape[1])(xs0, wat)
            zt = _mm_builder(xt0.shape[0], xt0.shape[1], was.shape[1])(xt0, was)
            mt_parts, ch = _messages_z(zs, inc, nt)
            g = _gather_diff(zt, inc, ne)
            at = _msg_post(xt0, wat, mt_parts, ch, deg, "attsig")
            as_ = _msg_post(xs0, was, g, None, None, "attsig")
            cnt_t = _sc_bincount(pos_t, n1)
            cnt_s = _sc_bincount(pos_s, e1)
            xt0 = _scatter_mean(at, nt, pos_t, cnt_t, n1)
            xs0 = _scatter_mean(as_, ne, pos_s, cnt_s, e1)
            ei_t, ew_t = edge_index_t1, edge_weight_t1
            ei_s, ew_s = edge_index_s1, edge_weight_s1
            inc = edge_index1
            nt, ne = n1, e1
            deg = _sc_bincount(inc.reshape(-1), nt)

    cnt_nb = _sc_bincount(n_batch1, ngraph)
    cnt_sb = _sc_bincount(s_batch1, ngraph)
    g_s = _scatter_mean(xs, ne, s_batch1, cnt_sb, ngraph)
    g_t = _scatter_mean(xt, nt, n_batch1, cnt_nb, ngraph)
    xg = jnp.concatenate([g_s, g_t], axis=-1)

    wout = p["out_W"]
    bout = jnp.broadcast_to(p["out_b"][None, :], (8, wout.shape[1]))
    return _head_builder(wout.shape[0], wout.shape[1])(xg, wout, bout)
